# Initial kernel scaffold; baseline (speedup 1.0000x reference)
import jax
import jax.numpy as jnp
from jax.experimental import pallas as pl


def kernel(x, y, edge_index, Wq, bq, Wk, bk, Wv, bv, We, be, Wr, br, Wg, bg, ln_g, ln_b):
    raise NotImplementedError("write your pallas kernel here")


# speedup vs baseline: 22.6043x; 22.6043x over previous
import functools

import jax
import jax.numpy as jnp
from jax import lax
from jax.experimental import pallas as pl
from jax.experimental.pallas import tpu as pltpu
from jax.experimental.pallas import tpu_sc as plsc

_N = 10000
_E = 320000
_D = 128
_H = 8
_DH = 16
_SCALE = 1.0 / (_DH ** 0.5)

_NC = 2
_NS = 16
_NW = _NC * _NS

_EB = 40
_EPW = _E // _NW
_NBLK = _EPW // _EB

_NP = 10240
_RPT = _NP // _NS
_RCH = 40
_NCH = _RPT // _RCH
_ZR = _NP * _H // _D
_ZPT = _ZR // _NS

_EB2 = 400
_NBLK2 = _EPW // _EB2

_HIGH = jax.lax.Precision.HIGHEST


def _proj_body(x_ref, w_ref, b_ref, q_ref, k_ref, v_ref, r_ref):
    o = jnp.dot(x_ref[...], w_ref[...], preferred_element_type=jnp.float32,
                precision=_HIGH) + b_ref[...]
    q_ref[...] = o[:, 0 * _D:1 * _D]
    k_ref[...] = o[:, 1 * _D:2 * _D]
    v_ref[...] = o[:, 2 * _D:3 * _D]
    r_ref[...] = o[:, 3 * _D:4 * _D]


def _proj(x, wcat, bcat):
    blk = 1000
    grid = _N // blk
    fs = jax.ShapeDtypeStruct((_N, _D), jnp.float32)
    return pl.pallas_call(
        _proj_body,
        grid=(grid,),
        in_specs=[
            pl.BlockSpec((blk, _D), lambda i: (i, 0)),
            pl.BlockSpec((_D, 4 * _D), lambda i: (0, 0)),
            pl.BlockSpec((1, 4 * _D), lambda i: (0, 0)),
        ],
        out_specs=[pl.BlockSpec((blk, _D), lambda i: (i, 0))] * 4,
        out_shape=[fs, fs, fs, fs],
    )(x, wcat, bcat)


def _eproj_body(y_ref, w_ref, b_ref, e_ref):
    e_ref[...] = jnp.dot(y_ref[...], w_ref[...],
                         preferred_element_type=jnp.float32,
                         precision=_HIGH) + b_ref[...]


def _eproj(y, we, be):
    blk = 2000
    grid = _E // blk
    return pl.pallas_call(
        _eproj_body,
        grid=(grid,),
        in_specs=[
            pl.BlockSpec((blk, _D), lambda i: (i, 0)),
            pl.BlockSpec((_D, _D), lambda i: (0, 0)),
            pl.BlockSpec((1, _D), lambda i: (0, 0)),
        ],
        out_specs=pl.BlockSpec((blk, _D), lambda i: (i, 0)),
        out_shape=jax.ShapeDtypeStruct((_E, _D), jnp.float32),
    )(y, we, be)


def _edge_body(q_hbm, k_hbm, v_hbm, e_hbm, src_hbm, dst_hbm,
               esc_hbm, opart_hbm, zpart_hbm,
               src_s, dst_s, zrow_s, dst_v, qrows, krows, vrows, erows,
               wrow, wz, esc_v, zbuf, acc, zacc, sem_q, sem_k, sem_v):
    c = lax.axis_index("c")
    s = lax.axis_index("s")
    wid = c * _NS + s
    lane = lax.broadcasted_iota(jnp.int32, (16,), 0)
    msk8 = lane < 8
    nmsk8 = jnp.logical_not(msk8)
    zero16 = jnp.zeros((16,), jnp.float32)

    def zb(t, _):
        r = t // (_D // 16)
        j = t % (_D // 16)
        zbuf[r, pl.ds(j * 16, 16)] = zero16
        return 0
    lax.fori_loop(0, _RCH * (_D // 16), zb, 0)
    row0 = s * _RPT
    for ch in range(_NCH):
        pltpu.sync_copy(zbuf, acc.at[pl.ds(row0 + ch * _RCH, _RCH)])
    zrow0 = s * _ZPT
    pltpu.sync_copy(zbuf, zacc.at[pl.ds(zrow0, _ZPT)])
    plsc.subcore_barrier()

    base_t = wid * _EPW

    def blk_body(b, _):
        base = pl.multiple_of(base_t + b * _EB, 8)
        pltpu.sync_copy(src_hbm.at[pl.ds(base, _EB)], src_s)
        pltpu.sync_copy(dst_hbm.at[pl.ds(base, _EB)], dst_s)
        pltpu.sync_copy(dst_hbm.at[pl.ds(base, _EB)],
                        dst_v.at[pl.ds(0, _EB)])
        pltpu.sync_copy(e_hbm.at[pl.ds(base, _EB)], erows)
        cq = pltpu.async_copy(q_hbm.at[dst_s], qrows, sem_q)
        ck = pltpu.async_copy(k_hbm.at[src_s], krows, sem_k)
        cv = pltpu.async_copy(v_hbm.at[src_s], vrows, sem_v)
        for cc in range(_EB // 16 + (1 if _EB % 16 else 0)):
            dvc = dst_v[pl.ds(cc * 16, 16)]
            rem = _EB - cc * 16
            m = None if rem >= 16 else (lane < rem)
            plsc.store_scatter(zrow_s, [lane + cc * 16],
                               jax.lax.shift_right_logical(dvc, 4), mask=m)
        cq.wait()
        ck.wait()
        cv.wait()

        def score_body(p, _):
            sv = zero16
            for t in range(16):
                i = 2 * p + t // _H
                h = t % _H
                sl = pl.ds(h * _DH, _DH)
                sc = jnp.sum(qrows[i, sl] * (krows[i, sl] + erows[i, sl]))
                sv = jnp.where(lane == t, sc, sv)
            esc_v[pl.ds(8 + p * 16, 16)] = jnp.exp(sv)
            return 0
        lax.fori_loop(0, _EB // 2, score_body, 0)

        def w_body(p, _):
            i0 = 2 * p
            i1 = 2 * p + 1
            eva = esc_v[pl.ds(p * 16, 16)]
            evb = esc_v[pl.ds(8 + p * 16, 16)]
            evc = esc_v[pl.ds(16 + p * 16, 16)]
            for h in range(_H):
                sl = pl.ds(h * _DH, _DH)
                wrow[i0, sl] = evb[h] * (vrows[i0, sl] + erows[i0, sl])
                wrow[i1, sl] = evc[h] * (vrows[i1, sl] + erows[i1, sl])
            dvv = dst_v[pl.ds(2 * p, 16)]
            d0 = dvv[0]
            d1 = dvv[1]
            for cz in range(_D // 16):
                wz[i0, pl.ds(cz * 16, 16)] = zero16
                wz[i1, pl.ds(cz * 16, 16)] = zero16
            c0 = (d0 & 15) >> 1
            c1 = (d1 & 15) >> 1
            val0 = jnp.where((d0 & 1) == 0, jnp.where(msk8, evb, 0.0),
                             jnp.where(nmsk8, eva, 0.0))
            val1 = jnp.where((d1 & 1) == 0, jnp.where(msk8, evc, 0.0),
                             jnp.where(nmsk8, evb, 0.0))
            wz[i0, pl.ds(c0 * 16, 16)] = val0
            wz[i1, pl.ds(c1 * 16, 16)] = val1
            return 0
        lax.fori_loop(0, _EB // 2, w_body, 0)

        pltpu.sync_copy(esc_v.at[pl.ds(8, _EB * _H)],
                        esc_hbm.at[pl.ds(base * _H, _EB * _H)])
        pltpu.sync_copy(wrow, acc.at[dst_s], add=True)
        pltpu.sync_copy(wz, zacc.at[zrow_s], add=True)
        return 0

    lax.fori_loop(0, _NBLK, blk_body, 0)

    plsc.subcore_barrier()
    for ch in range(_NCH):
        r0 = row0 + ch * _RCH
        pltpu.sync_copy(acc.at[pl.ds(r0, _RCH)], zbuf)
        pltpu.sync_copy(zbuf, opart_hbm.at[c, pl.ds(r0, _RCH)])
    pltpu.sync_copy(zacc.at[pl.ds(zrow0, _ZPT)], zbuf)
    pltpu.sync_copy(zbuf, zpart_hbm.at[c, pl.ds(zrow0, _ZPT)])


def _edge_pass(q, k, v, e, src, dst):
    mesh = plsc.VectorSubcoreMesh(core_axis_name="c", subcore_axis_name="s",
                                  num_cores=_NC, num_subcores=_NS)
    fn = pl.kernel(
        _edge_body,
        out_type=[
            jax.ShapeDtypeStruct((_E * _H,), jnp.float32),
            jax.ShapeDtypeStruct((_NC, _NP, _D), jnp.float32),
            jax.ShapeDtypeStruct((_NC, _ZR, _D), jnp.float32),
        ],
        mesh=mesh,
        scratch_types=[
            pltpu.VMEM((_EB,), jnp.int32),
            pltpu.VMEM((_EB,), jnp.int32),
            pltpu.VMEM((_EB,), jnp.int32),
            pltpu.VMEM((_EB + 16,), jnp.int32),
            pltpu.VMEM((_EB, _D), jnp.float32),
            pltpu.VMEM((_EB, _D), jnp.float32),
            pltpu.VMEM((_EB, _D), jnp.float32),
            pltpu.VMEM((_EB, _D), jnp.float32),
            pltpu.VMEM((_EB, _D), jnp.float32),
            pltpu.VMEM((_EB, _D), jnp.float32),
            pltpu.VMEM((8 + _EB * _H + 16,), jnp.float32),
            pltpu.VMEM((_RCH, _D), jnp.float32),
            pltpu.VMEM_SHARED((_NP, _D), jnp.float32),
            pltpu.VMEM_SHARED((_ZR, _D), jnp.float32),
            pltpu.SemaphoreType.DMA,
            pltpu.SemaphoreType.DMA,
            pltpu.SemaphoreType.DMA,
        ],
        compiler_params=pltpu.CompilerParams(needs_layout_passes=False),
    )
    return fn(q, k, v, e, src, dst)


def _fin_body(op_ref, zp_ref, r_ref, wgh_ref, wgr_ref, bg_ref, lng_ref,
              lnb_ref, out_ref, zinv_ref):
    blk = op_ref.shape[1]
    osum = op_ref[0] + op_ref[1]
    z = zp_ref[0] + zp_ref[1]
    zinv = _SCALE / (1e-8 + z)
    zinv_ref[...] = zinv
    h = (osum.reshape(blk, _H, _DH) * zinv[:, :, None]).reshape(blk, _D)
    r = r_ref[...]
    g = jnp.dot(h, wgh_ref[...], preferred_element_type=jnp.float32,
                precision=_HIGH)
    g += jnp.dot(r, wgr_ref[...], preferred_element_type=jnp.float32,
                 precision=_HIGH)
    b = jax.nn.sigmoid(g + bg_ref[...])
    hb = h - b * h + b * r
    mu = jnp.mean(hb, axis=1, keepdims=True)
    var = jnp.mean((hb - mu) ** 2, axis=1, keepdims=True)
    ln = (hb - mu) / jnp.sqrt(var + 1e-5) * lng_ref[...] + lnb_ref[...]
    out_ref[...] = jnp.where(ln >= 0, ln, 0.01 * ln)


def _finalize(opart, zpart, r, wgh, wgr, bg, lng, lnb):
    blk = 2000
    grid = _N // blk
    return pl.pallas_call(
        _fin_body,
        grid=(grid,),
        in_specs=[
            pl.BlockSpec((_NC, blk, _D), lambda i: (0, i, 0)),
            pl.BlockSpec((_NC, blk, _H), lambda i: (0, i, 0)),
            pl.BlockSpec((blk, _D), lambda i: (i, 0)),
            pl.BlockSpec((_D, _D), lambda i: (0, 0)),
            pl.BlockSpec((_D, _D), lambda i: (0, 0)),
            pl.BlockSpec((1, _D), lambda i: (0, 0)),
            pl.BlockSpec((1, _D), lambda i: (0, 0)),
            pl.BlockSpec((1, _D), lambda i: (0, 0)),
        ],
        out_specs=[
            pl.BlockSpec((blk, _D), lambda i: (i, 0)),
            pl.BlockSpec((blk, _H), lambda i: (i, 0)),
        ],
        out_shape=[
            jax.ShapeDtypeStruct((_N, _D), jnp.float32),
            jax.ShapeDtypeStruct((_N, _H), jnp.float32),
        ],
    )(opart, zpart, r, wgh, wgr, bg, lng, lnb)


def _norm_body(esc_hbm, dst_hbm, zinv_hbm, out_hbm, zv, dst_v, esc_v, nrm_v):
    c = lax.axis_index("c")
    s = lax.axis_index("s")
    wid = c * _NS + s
    pltpu.sync_copy(zinv_hbm, zv)
    lane = lax.broadcasted_iota(jnp.int32, (16,), 0)
    msk8 = lane < 8
    col = lane & 7
    base_t = wid * _EPW

    def blk_body(b, _):
        base = pl.multiple_of(base_t + b * _EB2, 8)
        pltpu.sync_copy(dst_hbm.at[pl.ds(base, _EB2)],
                        dst_v.at[pl.ds(0, _EB2)])
        pltpu.sync_copy(esc_hbm.at[pl.ds(base * _H, _EB2 * _H)], esc_v)

        def pair(j, _):
            dv = dst_v[pl.ds(2 * j, 16)]
            row = jnp.where(msk8, dv[0], dv[1])
            zi = plsc.load_gather(zv, [row * _H + col])
            sl = pl.ds(j * 16, 16)
            nrm_v[sl] = esc_v[sl] * zi
            return 0
        lax.fori_loop(0, _EB2 * _H // 16, pair, 0)
        pltpu.sync_copy(nrm_v, out_hbm.at[pl.ds(base * _H, _EB2 * _H)])
        return 0

    lax.fori_loop(0, _NBLK2, blk_body, 0)


def _norm_pass(esc, dst, zinv):
    mesh = plsc.VectorSubcoreMesh(core_axis_name="c", subcore_axis_name="s",
                                  num_cores=_NC, num_subcores=_NS)
    fn = pl.kernel(
        _norm_body,
        out_type=jax.ShapeDtypeStruct((_E * _H,), jnp.float32),
        mesh=mesh,
        scratch_types=[
            pltpu.VMEM((_N * _H,), jnp.float32),
            pltpu.VMEM((_EB2 + 16,), jnp.int32),
            pltpu.VMEM((_EB2 * _H,), jnp.float32),
            pltpu.VMEM((_EB2 * _H,), jnp.float32),
        ],
        compiler_params=pltpu.CompilerParams(needs_layout_passes=False),
    )
    return fn(esc, dst, zinv)


def kernel(x, y, edge_index, Wq, bq, Wk, bk, Wv, bv, We, be, Wr, br,
           Wg, bg, ln_g, ln_b):
    src = edge_index[0]
    dst = edge_index[1]
    wcat = jnp.concatenate([Wq, Wk, Wv, Wr], axis=1)
    bcat = jnp.concatenate([bq, bk, bv, br]).reshape(1, 4 * _D)
    q, k, v, r = _proj(x, wcat, bcat)
    e = _eproj(y, We, be.reshape(1, _D))
    esc, opart, zpart = _edge_pass(q, k, v, e, src, dst)
    zpart = zpart.reshape(_NC, _NP, _H)
    wgh = Wg[0:_D] + Wg[2 * _D:3 * _D]
    wgr = Wg[_D:2 * _D] - Wg[2 * _D:3 * _D]
    out, zinv = _finalize(opart, zpart, r, wgh, wgr, bg.reshape(1, _D),
                          ln_g.reshape(1, _D), ln_b.reshape(1, _D))
    nrm = _norm_pass(esc, dst, zinv.reshape(-1))
    return out, nrm.reshape(_E, _H, 1)


# speedup vs baseline: 33.4498x; 1.4798x over previous
import functools

import jax
import jax.numpy as jnp
from jax import lax
from jax.experimental import pallas as pl
from jax.experimental.pallas import tpu as pltpu
from jax.experimental.pallas import tpu_sc as plsc

_N = 10000
_E = 320000
_D = 128
_H = 8
_DH = 16
_SCALE = 1.0 / (_DH ** 0.5)

_NC = 2
_NS = 16
_NW = _NC * _NS

_EB = 40
_EPW = _E // _NW
_NBLK = _EPW // _EB

_NP = 10240
_RPT = _NP // _NS
_ZR = _NP * _H // _D
_ZPT = _ZR // _NS

_EB2 = 400
_NBLK2 = _EPW // _EB2

_HIGH = jax.lax.Precision.HIGHEST


def _cols_ab():
    ar = jnp.arange(_D // 2)
    w = ar // _DH
    d = ar % _DH
    cols_a = 32 * w + d
    return cols_a, cols_a + _DH


def _rne16(x):
    xi = lax.bitcast_convert_type(x, jnp.int32)
    return lax.shift_right_logical(
        xi + 0x7FFF + (lax.shift_right_logical(xi, 16) & 1), 16)


def _pack(a, b):
    return _rne16(a) | lax.shift_left(_rne16(b), 16)


def _proj_body(x_ref, w_ref, b_ref, q_ref, kv_ref, r_ref):
    o = jnp.dot(x_ref[...], w_ref[...], preferred_element_type=jnp.float32,
                precision=_HIGH) + b_ref[...]
    hd = _D // 2
    q_ref[:, 0:hd] = _pack(o[:, 0:hd], o[:, hd:2 * hd])
    q_ref[:, hd:2 * hd] = jnp.zeros((o.shape[0], hd), jnp.int32)
    kv_ref[:, 0:hd] = _pack(o[:, 2 * hd:3 * hd], o[:, 3 * hd:4 * hd])
    kv_ref[:, hd:2 * hd] = _pack(o[:, 4 * hd:5 * hd], o[:, 5 * hd:6 * hd])
    r_ref[...] = o[:, 6 * hd:8 * hd]


def _proj(x, wcat, bcat):
    blk = 1000
    grid = _N // blk
    return pl.pallas_call(
        _proj_body,
        grid=(grid,),
        in_specs=[
            pl.BlockSpec((blk, _D), lambda i: (i, 0)),
            pl.BlockSpec((_D, 4 * _D), lambda i: (0, 0)),
            pl.BlockSpec((1, 4 * _D), lambda i: (0, 0)),
        ],
        out_specs=[
            pl.BlockSpec((blk, _D), lambda i: (i, 0)),
            pl.BlockSpec((blk, _D), lambda i: (i, 0)),
            pl.BlockSpec((blk, _D), lambda i: (i, 0)),
        ],
        out_shape=[
            jax.ShapeDtypeStruct((_N, _D), jnp.int32),
            jax.ShapeDtypeStruct((_N, _D), jnp.int32),
            jax.ShapeDtypeStruct((_N, _D), jnp.float32),
        ],
    )(x, wcat, bcat)


def _eproj_body(y_ref, w_ref, b_ref, e_ref):
    o = jnp.dot(y_ref[...], w_ref[...], preferred_element_type=jnp.float32,
                precision=_HIGH) + b_ref[...]
    hd = _D // 2
    e_ref[...] = _pack(o[:, 0:hd], o[:, hd:2 * hd])


def _eproj(y, we, be):
    blk = 2000
    grid = _E // blk
    return pl.pallas_call(
        _eproj_body,
        grid=(grid,),
        in_specs=[
            pl.BlockSpec((blk, _D), lambda i: (i, 0)),
            pl.BlockSpec((_D, _D), lambda i: (0, 0)),
            pl.BlockSpec((1, _D), lambda i: (0, 0)),
        ],
        out_specs=pl.BlockSpec((blk, _D // 2), lambda i: (i, 0)),
        out_shape=jax.ShapeDtypeStruct((_E, _D // 2), jnp.int32),
    )(y, we, be)


def _edge_body(q_hbm, kv_hbm, e_hbm, src_hbm, dst_hbm, zero_hbm,
               esc_hbm, opart_hbm, zpart_hbm,
               src_s0, dst_s0, dst_v0, src_s1, dst_s1, dst_v1,
               qr0, kvr0, er0, qr1, kvr1, er1,
               esc0, esc1, wrow, dsto, wz, zrowo,
               acc, zacc, sem_i0, sem_i1, sem_o0, sem_o1, sem_z):
    c = lax.axis_index("c")
    s = lax.axis_index("s")
    wid = c * _NS + s
    lane = lax.broadcasted_iota(jnp.int32, (16,), 0)
    msk8 = lane < 8
    nmsk8 = jnp.logical_not(msk8)
    zero16 = jnp.zeros((16,), jnp.float32)
    hi_mask = jnp.full((16,), -65536, jnp.int32)

    idx_sets = ((src_s0, dst_s0, dst_v0), (src_s1, dst_s1, dst_v1))
    in_sets = ((qr0, kvr0, er0), (qr1, kvr1, er1))
    out_sets = (esc0, esc1)
    in_sems = (sem_i0, sem_i1)
    out_sems = (sem_o0, sem_o1)

    row0 = s * _RPT
    zr0 = s * _ZPT
    pltpu.sync_copy(zero_hbm, acc.at[pl.ds(row0, _RPT)])
    pltpu.sync_copy(zero_hbm.at[pl.ds(0, _ZPT)], zacc.at[pl.ds(zr0, _ZPT)])
    plsc.subcore_barrier()

    base_t = wid * _EPW

    def stage_idx(db, b):
        src_s, dst_s, dst_v = idx_sets[db]
        base = pl.multiple_of(base_t + b * _EB, 8)
        pltpu.sync_copy(src_hbm.at[pl.ds(base, _EB)], src_s)
        pltpu.sync_copy(dst_hbm.at[pl.ds(base, _EB)], dst_s)
        pltpu.sync_copy(dst_hbm.at[pl.ds(base, _EB)],
                        dst_v.at[pl.ds(0, _EB)])

    def issue_gathers(db, b):
        src_s, dst_s, _ = idx_sets[db]
        qr, kvr, er = in_sets[db]
        base = pl.multiple_of(base_t + b * _EB, 8)
        pltpu.make_async_copy(e_hbm.at[pl.ds(base, _EB)], er,
                              in_sems[db]).start()
        pltpu.make_async_copy(q_hbm.at[dst_s], qr, in_sems[db]).start()
        pltpu.make_async_copy(kv_hbm.at[src_s], kvr, in_sems[db]).start()

    def wait_gathers(db):
        qr, kvr, er = in_sets[db]
        pltpu.make_async_copy(e_hbm.at[pl.ds(0, _EB)], er,
                              in_sems[db]).wait()
        pltpu.make_async_copy(q_hbm.at[pl.ds(0, _EB)], qr,
                              in_sems[db]).wait()
        pltpu.make_async_copy(kv_hbm.at[pl.ds(0, _EB)], kvr,
                              in_sems[db]).wait()

    def drain_outs(db):
        pltpu.make_async_copy(out_sets[db].at[pl.ds(8, _EB * _H)],
                              esc_hbm.at[pl.ds(0, _EB * _H)],
                              out_sems[db]).wait()

    def drain_scat():
        pltpu.make_async_copy(wrow, acc.at[pl.ds(0, _EB)], sem_z).wait()
        pltpu.make_async_copy(wz, zacc.at[pl.ds(0, _EB)], sem_z).wait()

    def issue_outs(db, b):
        base = pl.multiple_of(base_t + b * _EB, 8)
        pltpu.make_async_copy(out_sets[db].at[pl.ds(8, _EB * _H)],
                              esc_hbm.at[pl.ds(base * _H, _EB * _H)],
                              out_sems[db]).start()
        pltpu.make_async_copy(wrow, acc.at[dsto], sem_z).start(add=True)
        pltpu.make_async_copy(wz, zacc.at[zrowo], sem_z).start(add=True)

    def unpk(ref, i, base_col, w):
        b32 = ref[i, pl.ds(base_col + w * 16, 16)]
        lo = plsc.bitcast(lax.shift_left(b32, 16), jnp.float32)
        hi = plsc.bitcast(lax.bitwise_and(b32, hi_mask), jnp.float32)
        return lo, hi

    def compute(db):
        _, _, dst_v = idx_sets[db]
        qr, kvr, er = in_sets[db]
        esc_v = out_sets[db]

        for cc in range(3):
            dvc = dst_v[pl.ds(cc * 16, 16)]
            rem = _EB - cc * 16
            m = None if rem >= 16 else (lane < rem)
            plsc.store_scatter(dsto, [lane + cc * 16], dvc, mask=m)
            plsc.store_scatter(zrowo, [lane + cc * 16],
                               lax.shift_right_logical(dvc, 4), mask=m)

        def pair_body(p, _):
            sv = zero16
            for eo in (0, 1):
                i = 2 * p + eo
                for w in range(_H // 2):
                    q_lo, q_hi = unpk(qr, i, 0, w)
                    k_lo, k_hi = unpk(kvr, i, 0, w)
                    e_lo, e_hi = unpk(er, i, 0, w)
                    s_lo = jnp.sum(q_lo * (k_lo + e_lo))
                    s_hi = jnp.sum(q_hi * (k_hi + e_hi))
                    sv = jnp.where(lane == eo * 8 + 2 * w, s_lo, sv)
                    sv = jnp.where(lane == eo * 8 + 2 * w + 1, s_hi, sv)
            es = jnp.exp(sv)
            esc_v[pl.ds(8 + p * 16, 16)] = es
            i0 = 2 * p
            i1 = 2 * p + 1
            for eo in (0, 1):
                i = 2 * p + eo
                for w in range(_H // 2):
                    v_lo, v_hi = unpk(kvr, i, _D // 2, w)
                    e_lo, e_hi = unpk(er, i, 0, w)
                    wrow[i, pl.ds((2 * w) * _DH, _DH)] = (
                        es[eo * 8 + 2 * w] * (v_lo + e_lo))
                    wrow[i, pl.ds((2 * w + 1) * _DH, _DH)] = (
                        es[eo * 8 + 2 * w + 1] * (v_hi + e_hi))
            eva = esc_v[pl.ds(p * 16, 16)]
            evc = esc_v[pl.ds(16 + p * 16, 16)]
            dvv = dst_v[pl.ds(2 * p, 16)]
            d0 = dvv[0]
            d1 = dvv[1]
            for cz in range(_D // 16):
                wz[i0, pl.ds(cz * 16, 16)] = zero16
                wz[i1, pl.ds(cz * 16, 16)] = zero16
            c0 = (d0 & 15) >> 1
            c1 = (d1 & 15) >> 1
            val0 = jnp.where((d0 & 1) == 0, jnp.where(msk8, es, 0.0),
                             jnp.where(nmsk8, eva, 0.0))
            val1 = jnp.where((d1 & 1) == 0, jnp.where(msk8, evc, 0.0),
                             jnp.where(nmsk8, es, 0.0))
            wz[i0, pl.ds(c0 * 16, 16)] = val0
            wz[i1, pl.ds(c1 * 16, 16)] = val1
            return 0
        lax.fori_loop(0, _EB // 2, pair_body, 0)

    stage_idx(0, 0)
    issue_gathers(0, 0)

    def iter_body(g, _):
        for db in (0, 1):
            b = 2 * g + db
            nx = 1 - db

            @pl.when(b + 1 < _NBLK)
            def _():
                stage_idx(nx, b + 1)
                issue_gathers(nx, b + 1)

            wait_gathers(db)

            @pl.when(b >= 2)
            def _():
                drain_outs(db)

            @pl.when(b >= 1)
            def _():
                drain_scat()

            compute(db)
            issue_outs(db, b)
        return 0

    lax.fori_loop(0, _NBLK // 2, iter_body, 0)
    drain_outs(0)
    drain_outs(1)
    drain_scat()

    plsc.subcore_barrier()
    pltpu.sync_copy(acc.at[pl.ds(row0, _RPT)],
                    opart_hbm.at[c, pl.ds(row0, _RPT)])
    pltpu.sync_copy(zacc.at[pl.ds(zr0, _ZPT)],
                    zpart_hbm.at[c, pl.ds(zr0, _ZPT)])


def _edge_pass(q, kv, e, src, dst, zero):
    mesh = plsc.VectorSubcoreMesh(core_axis_name="c", subcore_axis_name="s",
                                  num_cores=_NC, num_subcores=_NS)
    idx_t = [pltpu.VMEM((_EB,), jnp.int32),
             pltpu.VMEM((_EB,), jnp.int32),
             pltpu.VMEM((_EB + 16,), jnp.int32)]
    in_t = [pltpu.VMEM((_EB, _D), jnp.int32),
            pltpu.VMEM((_EB, _D), jnp.int32),
            pltpu.VMEM((_EB, _D // 2), jnp.int32)]
    out_t = [pltpu.VMEM((8 + _EB * _H + 16,), jnp.float32)]
    fn = pl.kernel(
        _edge_body,
        out_type=[
            jax.ShapeDtypeStruct((_E * _H,), jnp.float32),
            jax.ShapeDtypeStruct((_NC, _NP, _D), jnp.float32),
            jax.ShapeDtypeStruct((_NC, _ZR, _D), jnp.float32),
        ],
        mesh=mesh,
        scratch_types=(idx_t + idx_t + in_t + in_t + out_t + out_t + [
            pltpu.VMEM((_EB, _D), jnp.float32),
            pltpu.VMEM((_EB,), jnp.int32),
            pltpu.VMEM((_EB, _D), jnp.float32),
            pltpu.VMEM((_EB,), jnp.int32),
            pltpu.VMEM_SHARED((_NP, _D), jnp.float32),
            pltpu.VMEM_SHARED((_ZR, _D), jnp.float32),
            pltpu.SemaphoreType.DMA,
            pltpu.SemaphoreType.DMA,
            pltpu.SemaphoreType.DMA,
            pltpu.SemaphoreType.DMA,
            pltpu.SemaphoreType.DMA,
        ]),
        compiler_params=pltpu.CompilerParams(needs_layout_passes=False),
    )
    return fn(q, kv, e, src, dst, zero)


def _fin_body(op_ref, zp_ref, r_ref, wgh_ref, wgr_ref, bg_ref, lng_ref,
              lnb_ref, out_ref, zinv_ref):
    blk = op_ref.shape[1]
    osum = op_ref[0] + op_ref[1]
    z = zp_ref[0] + zp_ref[1]
    zinv = _SCALE / (1e-8 + z)
    zinv_ref[...] = zinv
    h = (osum.reshape(blk, _H, _DH) * zinv[:, :, None]).reshape(blk, _D)
    r = r_ref[...]
    g = jnp.dot(h, wgh_ref[...], preferred_element_type=jnp.float32,
                precision=_HIGH)
    g += jnp.dot(r, wgr_ref[...], preferred_element_type=jnp.float32,
                 precision=_HIGH)
    b = jax.nn.sigmoid(g + bg_ref[...])
    hb = h - b * h + b * r
    mu = jnp.mean(hb, axis=1, keepdims=True)
    var = jnp.mean((hb - mu) ** 2, axis=1, keepdims=True)
    ln = (hb - mu) / jnp.sqrt(var + 1e-5) * lng_ref[...] + lnb_ref[...]
    out_ref[...] = jnp.where(ln >= 0, ln, 0.01 * ln)


def _finalize(opart, zpart, r, wgh, wgr, bg, lng, lnb):
    blk = 2000
    grid = _N // blk
    return pl.pallas_call(
        _fin_body,
        grid=(grid,),
        in_specs=[
            pl.BlockSpec((_NC, blk, _D), lambda i: (0, i, 0)),
            pl.BlockSpec((_NC, blk, _H), lambda i: (0, i, 0)),
            pl.BlockSpec((blk, _D), lambda i: (i, 0)),
            pl.BlockSpec((_D, _D), lambda i: (0, 0)),
            pl.BlockSpec((_D, _D), lambda i: (0, 0)),
            pl.BlockSpec((1, _D), lambda i: (0, 0)),
            pl.BlockSpec((1, _D), lambda i: (0, 0)),
            pl.BlockSpec((1, _D), lambda i: (0, 0)),
        ],
        out_specs=[
            pl.BlockSpec((blk, _D), lambda i: (i, 0)),
            pl.BlockSpec((blk, _H), lambda i: (i, 0)),
        ],
        out_shape=[
            jax.ShapeDtypeStruct((_N, _D), jnp.float32),
            jax.ShapeDtypeStruct((_N, _H), jnp.float32),
        ],
    )(opart, zpart, r, wgh, wgr, bg, lng, lnb)


def _norm_body(esc_hbm, dst_hbm, zinv_hbm, out_hbm, zv, dst_v, esc_v, nrm_v):
    c = lax.axis_index("c")
    s = lax.axis_index("s")
    wid = c * _NS + s
    pltpu.sync_copy(zinv_hbm, zv)
    lane = lax.broadcasted_iota(jnp.int32, (16,), 0)
    msk8 = lane < 8
    col = lane & 7
    base_t = wid * _EPW

    def blk_body(b, _):
        base = pl.multiple_of(base_t + b * _EB2, 8)
        pltpu.sync_copy(dst_hbm.at[pl.ds(base, _EB2)],
                        dst_v.at[pl.ds(0, _EB2)])
        pltpu.sync_copy(esc_hbm.at[pl.ds(base * _H, _EB2 * _H)], esc_v)

        def pair(j, _):
            dv = dst_v[pl.ds(2 * j, 16)]
            row = jnp.where(msk8, dv[0], dv[1])
            zi = plsc.load_gather(zv, [row * _H + col])
            sl = pl.ds(j * 16, 16)
            nrm_v[sl] = esc_v[sl] * zi
            return 0
        lax.fori_loop(0, _EB2 * _H // 16, pair, 0)
        pltpu.sync_copy(nrm_v, out_hbm.at[pl.ds(base * _H, _EB2 * _H)])
        return 0

    lax.fori_loop(0, _NBLK2, blk_body, 0)


def _norm_pass(esc, dst, zinv):
    mesh = plsc.VectorSubcoreMesh(core_axis_name="c", subcore_axis_name="s",
                                  num_cores=_NC, num_subcores=_NS)
    fn = pl.kernel(
        _norm_body,
        out_type=jax.ShapeDtypeStruct((_E * _H,), jnp.float32),
        mesh=mesh,
        scratch_types=[
            pltpu.VMEM((_N * _H,), jnp.float32),
            pltpu.VMEM((_EB2 + 16,), jnp.int32),
            pltpu.VMEM((_EB2 * _H,), jnp.float32),
            pltpu.VMEM((_EB2 * _H,), jnp.float32),
        ],
        compiler_params=pltpu.CompilerParams(needs_layout_passes=False),
    )
    return fn(esc, dst, zinv)


def kernel(x, y, edge_index, Wq, bq, Wk, bk, Wv, bv, We, be, Wr, br,
           Wg, bg, ln_g, ln_b):
    src = edge_index[0]
    dst = edge_index[1]
    ca, cb = _cols_ab()
    wcat = jnp.concatenate([Wq[:, ca], Wq[:, cb], Wk[:, ca], Wk[:, cb],
                            Wv[:, ca], Wv[:, cb], Wr], axis=1)
    bcat = jnp.concatenate([bq[ca], bq[cb], bk[ca], bk[cb], bv[ca], bv[cb],
                            br]).reshape(1, 4 * _D)
    q, kv, r = _proj(x, wcat, bcat)
    e = _eproj(y, jnp.concatenate([We[:, ca], We[:, cb]], axis=1),
               jnp.concatenate([be[ca], be[cb]]).reshape(1, _D))
    zero = jnp.zeros((_RPT, _D), jnp.float32)
    esc, opart, zpart = _edge_pass(q, kv, e, src, dst, zero)
    zpart = zpart.reshape(_NC, _NP, _H)
    wgh = Wg[0:_D] + Wg[2 * _D:3 * _D]
    wgr = Wg[_D:2 * _D] - Wg[2 * _D:3 * _D]
    out, zinv = _finalize(opart, zpart, r, wgh, wgr, bg.reshape(1, _D),
                          ln_g.reshape(1, _D), ln_b.reshape(1, _D))
    nrm = _norm_pass(esc, dst, zinv.reshape(-1))
    return out, nrm.reshape(_E, _H, 1)


# speedup vs baseline: 35.0676x; 1.0484x over previous
import functools

import jax
import jax.numpy as jnp
from jax import lax
from jax.experimental import pallas as pl
from jax.experimental.pallas import tpu as pltpu
from jax.experimental.pallas import tpu_sc as plsc

_N = 10000
_E = 320000
_D = 128
_H = 8
_DH = 16
_SCALE = 1.0 / (_DH ** 0.5)

_NC = 2
_NS = 16
_NW = _NC * _NS

_EB = 40
_EPW = _E // _NW
_NBLK = _EPW // _EB

_NP = 10240
_RPT = _NP // _NS
_ZR = _NP * _H // _D
_ZPT = _ZR // _NS

_EB2 = 400
_NBLK2 = _EPW // _EB2

_HIGH = jax.lax.Precision.DEFAULT


def _cols_ab():
    ar = jnp.arange(_D // 2)
    w = ar // _DH
    d = ar % _DH
    cols_a = 32 * w + d
    return cols_a, cols_a + _DH


def _rne16(x):
    xi = lax.bitcast_convert_type(x, jnp.int32)
    return lax.shift_right_logical(
        xi + 0x7FFF + (lax.shift_right_logical(xi, 16) & 1), 16)


def _pack(a, b):
    return _rne16(a) | lax.shift_left(_rne16(b), 16)


def _proj_body(x_ref, w_ref, b_ref, q_ref, kv_ref, r_ref):
    o = jnp.dot(x_ref[...], w_ref[...], preferred_element_type=jnp.float32,
                precision=_HIGH) + b_ref[...]
    hd = _D // 2
    q_ref[:, 0:hd] = _pack(o[:, 0:hd], o[:, hd:2 * hd])
    q_ref[:, hd:2 * hd] = jnp.zeros((o.shape[0], hd), jnp.int32)
    kv_ref[:, 0:hd] = _pack(o[:, 2 * hd:3 * hd], o[:, 3 * hd:4 * hd])
    kv_ref[:, hd:2 * hd] = _pack(o[:, 4 * hd:5 * hd], o[:, 5 * hd:6 * hd])
    r_ref[...] = o[:, 6 * hd:8 * hd]


def _proj(x, wcat, bcat):
    blk = 1000
    grid = _N // blk
    return pl.pallas_call(
        _proj_body,
        grid=(grid,),
        in_specs=[
            pl.BlockSpec((blk, _D), lambda i: (i, 0)),
            pl.BlockSpec((_D, 4 * _D), lambda i: (0, 0)),
            pl.BlockSpec((1, 4 * _D), lambda i: (0, 0)),
        ],
        out_specs=[
            pl.BlockSpec((blk, _D), lambda i: (i, 0)),
            pl.BlockSpec((blk, _D), lambda i: (i, 0)),
            pl.BlockSpec((blk, _D), lambda i: (i, 0)),
        ],
        out_shape=[
            jax.ShapeDtypeStruct((_N, _D), jnp.int32),
            jax.ShapeDtypeStruct((_N, _D), jnp.int32),
            jax.ShapeDtypeStruct((_N, _D), jnp.float32),
        ],
    )(x, wcat, bcat)


def _eproj_body(y_ref, w_ref, b_ref, e_ref):
    o = jnp.dot(y_ref[...], w_ref[...], preferred_element_type=jnp.float32,
                precision=_HIGH) + b_ref[...]
    hd = _D // 2
    e_ref[...] = _pack(o[:, 0:hd], o[:, hd:2 * hd])


def _eproj(y, we, be):
    blk = 2000
    grid = _E // blk
    return pl.pallas_call(
        _eproj_body,
        grid=(grid,),
        in_specs=[
            pl.BlockSpec((blk, _D), lambda i: (i, 0)),
            pl.BlockSpec((_D, _D), lambda i: (0, 0)),
            pl.BlockSpec((1, _D), lambda i: (0, 0)),
        ],
        out_specs=pl.BlockSpec((blk, _D // 2), lambda i: (i, 0)),
        out_shape=jax.ShapeDtypeStruct((_E, _D // 2), jnp.int32),
    )(y, we, be)


def _edge_body(q_hbm, kv_hbm, e_hbm, ei_hbm, zero_hbm,
               esc_hbm, opart_hbm, zpart_hbm,
               src_s0, dst_s0, dst_v0, src_s1, dst_s1, dst_v1,
               qr0, kvr0, er0, qr1, kvr1, er1,
               esc0, esc1, wrow, dsto, wz, zrowo,
               acc, zacc, sem_i0, sem_i1, sem_o0, sem_o1, sem_z):
    c = lax.axis_index("c")
    s = lax.axis_index("s")
    wid = c * _NS + s
    lane = lax.broadcasted_iota(jnp.int32, (16,), 0)
    msk8 = lane < 8
    nmsk8 = jnp.logical_not(msk8)
    zero16 = jnp.zeros((16,), jnp.float32)
    hi_mask = jnp.full((16,), -65536, jnp.int32)

    idx_sets = ((src_s0, dst_s0, dst_v0), (src_s1, dst_s1, dst_v1))
    in_sets = ((qr0, kvr0, er0), (qr1, kvr1, er1))
    out_sets = (esc0, esc1)
    in_sems = (sem_i0, sem_i1)
    out_sems = (sem_o0, sem_o1)

    row0 = s * _RPT
    zr0 = s * _ZPT
    pltpu.sync_copy(zero_hbm, acc.at[pl.ds(row0, _RPT)])
    pltpu.sync_copy(zero_hbm.at[pl.ds(0, _ZPT)], zacc.at[pl.ds(zr0, _ZPT)])
    plsc.subcore_barrier()

    base_t = wid * _EPW

    def stage_idx(db, b):
        src_s, dst_s, dst_v = idx_sets[db]
        base = pl.multiple_of(base_t + b * _EB, 8)
        pltpu.sync_copy(ei_hbm.at[pl.ds(base, _EB)], src_s)
        pltpu.sync_copy(ei_hbm.at[pl.ds(_E + base, _EB)], dst_s)
        pltpu.sync_copy(ei_hbm.at[pl.ds(_E + base, _EB)],
                        dst_v.at[pl.ds(0, _EB)])

    def issue_gathers(db, b):
        src_s, dst_s, _ = idx_sets[db]
        qr, kvr, er = in_sets[db]
        base = pl.multiple_of(base_t + b * _EB, 8)
        pltpu.make_async_copy(e_hbm.at[pl.ds(base, _EB)], er,
                              in_sems[db]).start()
        pltpu.make_async_copy(q_hbm.at[dst_s], qr, in_sems[db]).start()
        pltpu.make_async_copy(kv_hbm.at[src_s], kvr, in_sems[db]).start()

    def wait_gathers(db):
        qr, kvr, er = in_sets[db]
        pltpu.make_async_copy(e_hbm.at[pl.ds(0, _EB)], er,
                              in_sems[db]).wait()
        pltpu.make_async_copy(q_hbm.at[pl.ds(0, _EB)], qr,
                              in_sems[db]).wait()
        pltpu.make_async_copy(kv_hbm.at[pl.ds(0, _EB)], kvr,
                              in_sems[db]).wait()

    def drain_outs(db):
        pltpu.make_async_copy(out_sets[db].at[pl.ds(8, _EB * _H)],
                              esc_hbm.at[pl.ds(0, _EB * _H)],
                              out_sems[db]).wait()

    def drain_scat():
        pltpu.make_async_copy(wrow, acc.at[pl.ds(0, _EB)], sem_z).wait()
        pltpu.make_async_copy(wz, zacc.at[pl.ds(0, _EB)], sem_z).wait()

    def issue_outs(db, b):
        base = pl.multiple_of(base_t + b * _EB, 8)
        pltpu.make_async_copy(out_sets[db].at[pl.ds(8, _EB * _H)],
                              esc_hbm.at[pl.ds(base * _H, _EB * _H)],
                              out_sems[db]).start()
        pltpu.make_async_copy(wrow, acc.at[dsto], sem_z).start(add=True)
        pltpu.make_async_copy(wz, zacc.at[zrowo], sem_z).start(add=True)

    def unpk(ref, i, base_col, w):
        b32 = ref[i, pl.ds(base_col + w * 16, 16)]
        lo = plsc.bitcast(lax.shift_left(b32, 16), jnp.float32)
        hi = plsc.bitcast(lax.bitwise_and(b32, hi_mask), jnp.float32)
        return lo, hi

    def compute(db):
        _, _, dst_v = idx_sets[db]
        qr, kvr, er = in_sets[db]
        esc_v = out_sets[db]

        for cc in range(3):
            dvc = dst_v[pl.ds(cc * 16, 16)]
            rem = _EB - cc * 16
            m = None if rem >= 16 else (lane < rem)
            plsc.store_scatter(dsto, [lane + cc * 16], dvc, mask=m)
            plsc.store_scatter(zrowo, [lane + cc * 16],
                               lax.shift_right_logical(dvc, 4), mask=m)

        def pair_body(p, _):
            sv = zero16
            for eo in (0, 1):
                i = 2 * p + eo
                for w in range(_H // 2):
                    q_lo, q_hi = unpk(qr, i, 0, w)
                    k_lo, k_hi = unpk(kvr, i, 0, w)
                    e_lo, e_hi = unpk(er, i, 0, w)
                    s_lo = jnp.sum(q_lo * (k_lo + e_lo))
                    s_hi = jnp.sum(q_hi * (k_hi + e_hi))
                    sv = jnp.where(lane == eo * 8 + 2 * w, s_lo, sv)
                    sv = jnp.where(lane == eo * 8 + 2 * w + 1, s_hi, sv)
            es = jnp.exp(sv)
            esc_v[pl.ds(8 + p * 16, 16)] = es
            i0 = 2 * p
            i1 = 2 * p + 1
            for eo in (0, 1):
                i = 2 * p + eo
                for w in range(_H // 2):
                    v_lo, v_hi = unpk(kvr, i, _D // 2, w)
                    e_lo, e_hi = unpk(er, i, 0, w)
                    wrow[i, pl.ds((2 * w) * _DH, _DH)] = (
                        es[eo * 8 + 2 * w] * (v_lo + e_lo))
                    wrow[i, pl.ds((2 * w + 1) * _DH, _DH)] = (
                        es[eo * 8 + 2 * w + 1] * (v_hi + e_hi))
            eva = esc_v[pl.ds(p * 16, 16)]
            evc = esc_v[pl.ds(16 + p * 16, 16)]
            dvv = dst_v[pl.ds(2 * p, 16)]
            d0 = dvv[0]
            d1 = dvv[1]
            for cz in range(_D // 16):
                wz[i0, pl.ds(cz * 16, 16)] = zero16
                wz[i1, pl.ds(cz * 16, 16)] = zero16
            c0 = (d0 & 15) >> 1
            c1 = (d1 & 15) >> 1
            val0 = jnp.where((d0 & 1) == 0, jnp.where(msk8, es, 0.0),
                             jnp.where(nmsk8, eva, 0.0))
            val1 = jnp.where((d1 & 1) == 0, jnp.where(msk8, evc, 0.0),
                             jnp.where(nmsk8, es, 0.0))
            wz[i0, pl.ds(c0 * 16, 16)] = val0
            wz[i1, pl.ds(c1 * 16, 16)] = val1
            return 0
        lax.fori_loop(0, _EB // 2, pair_body, 0)

    stage_idx(0, 0)
    issue_gathers(0, 0)

    def iter_body(g, _):
        for db in (0, 1):
            b = 2 * g + db
            nx = 1 - db

            @pl.when(b + 1 < _NBLK)
            def _():
                stage_idx(nx, b + 1)
                issue_gathers(nx, b + 1)

            wait_gathers(db)

            @pl.when(b >= 2)
            def _():
                drain_outs(db)

            @pl.when(b >= 1)
            def _():
                drain_scat()

            compute(db)
            issue_outs(db, b)
        return 0

    lax.fori_loop(0, _NBLK // 2, iter_body, 0)
    drain_outs(0)
    drain_outs(1)
    drain_scat()

    plsc.subcore_barrier()
    pltpu.sync_copy(acc.at[pl.ds(row0, _RPT)],
                    opart_hbm.at[c, pl.ds(row0, _RPT)])
    pltpu.sync_copy(zacc.at[pl.ds(zr0, _ZPT)],
                    zpart_hbm.at[c, pl.ds(zr0, _ZPT)])


def _edge_pass(q, kv, e, ei, zero):
    mesh = plsc.VectorSubcoreMesh(core_axis_name="c", subcore_axis_name="s",
                                  num_cores=_NC, num_subcores=_NS)
    idx_t = [pltpu.VMEM((_EB,), jnp.int32),
             pltpu.VMEM((_EB,), jnp.int32),
             pltpu.VMEM((_EB + 16,), jnp.int32)]
    in_t = [pltpu.VMEM((_EB, _D), jnp.int32),
            pltpu.VMEM((_EB, _D), jnp.int32),
            pltpu.VMEM((_EB, _D // 2), jnp.int32)]
    out_t = [pltpu.VMEM((8 + _EB * _H + 16,), jnp.float32)]
    fn = pl.kernel(
        _edge_body,
        out_type=[
            jax.ShapeDtypeStruct((_E * _H,), jnp.float32),
            jax.ShapeDtypeStruct((_NC, _NP, _D), jnp.float32),
            jax.ShapeDtypeStruct((_NC, _ZR, _D), jnp.float32),
        ],
        mesh=mesh,
        scratch_types=(idx_t + idx_t + in_t + in_t + out_t + out_t + [
            pltpu.VMEM((_EB, _D), jnp.float32),
            pltpu.VMEM((_EB,), jnp.int32),
            pltpu.VMEM((_EB, _D), jnp.float32),
            pltpu.VMEM((_EB,), jnp.int32),
            pltpu.VMEM_SHARED((_NP, _D), jnp.float32),
            pltpu.VMEM_SHARED((_ZR, _D), jnp.float32),
            pltpu.SemaphoreType.DMA,
            pltpu.SemaphoreType.DMA,
            pltpu.SemaphoreType.DMA,
            pltpu.SemaphoreType.DMA,
            pltpu.SemaphoreType.DMA,
        ]),
        compiler_params=pltpu.CompilerParams(needs_layout_passes=False),
    )
    return fn(q, kv, e, ei, zero)


def _fin_body(op_ref, zp_ref, r_ref, wgh_ref, wgr_ref, bg_ref, lng_ref,
              lnb_ref, out_ref, zinv_ref):
    blk = op_ref.shape[1]
    osum = op_ref[0] + op_ref[1]
    z = zp_ref[0] + zp_ref[1]
    zinv = _SCALE / (1e-8 + z)
    zinv_ref[...] = zinv
    h = (osum.reshape(blk, _H, _DH) * zinv[:, :, None]).reshape(blk, _D)
    r = r_ref[...]
    g = jnp.dot(h, wgh_ref[...], preferred_element_type=jnp.float32,
                precision=_HIGH)
    g += jnp.dot(r, wgr_ref[...], preferred_element_type=jnp.float32,
                 precision=_HIGH)
    b = jax.nn.sigmoid(g + bg_ref[...])
    hb = h - b * h + b * r
    mu = jnp.mean(hb, axis=1, keepdims=True)
    var = jnp.mean((hb - mu) ** 2, axis=1, keepdims=True)
    ln = (hb - mu) / jnp.sqrt(var + 1e-5) * lng_ref[...] + lnb_ref[...]
    out_ref[...] = jnp.where(ln >= 0, ln, 0.01 * ln)


def _finalize(opart, zpart, r, wgh, wgr, bg, lng, lnb):
    blk = 2000
    grid = _N // blk
    return pl.pallas_call(
        _fin_body,
        grid=(grid,),
        in_specs=[
            pl.BlockSpec((_NC, blk, _D), lambda i: (0, i, 0)),
            pl.BlockSpec((_NC, blk, _H), lambda i: (0, i, 0)),
            pl.BlockSpec((blk, _D), lambda i: (i, 0)),
            pl.BlockSpec((_D, _D), lambda i: (0, 0)),
            pl.BlockSpec((_D, _D), lambda i: (0, 0)),
            pl.BlockSpec((1, _D), lambda i: (0, 0)),
            pl.BlockSpec((1, _D), lambda i: (0, 0)),
            pl.BlockSpec((1, _D), lambda i: (0, 0)),
        ],
        out_specs=[
            pl.BlockSpec((blk, _D), lambda i: (i, 0)),
            pl.BlockSpec((blk, _H), lambda i: (i, 0)),
        ],
        out_shape=[
            jax.ShapeDtypeStruct((_N, _D), jnp.float32),
            jax.ShapeDtypeStruct((_N, _H), jnp.float32),
        ],
    )(opart, zpart, r, wgh, wgr, bg, lng, lnb)


def _norm_body(esc_hbm, ei_hbm, zinv_hbm, out_hbm, zv, dst_v, esc_v, nrm_v):
    c = lax.axis_index("c")
    s = lax.axis_index("s")
    wid = c * _NS + s
    pltpu.sync_copy(zinv_hbm, zv)
    lane = lax.broadcasted_iota(jnp.int32, (16,), 0)
    msk8 = lane < 8
    col = lane & 7
    base_t = wid * _EPW

    def blk_body(b, _):
        base = pl.multiple_of(base_t + b * _EB2, 8)
        pltpu.sync_copy(ei_hbm.at[pl.ds(_E + base, _EB2)],
                        dst_v.at[pl.ds(0, _EB2)])
        pltpu.sync_copy(esc_hbm.at[pl.ds(base * _H, _EB2 * _H)], esc_v)

        def pair(j, _):
            dv = dst_v[pl.ds(2 * j, 16)]
            row = jnp.where(msk8, dv[0], dv[1])
            zi = plsc.load_gather(zv, [row * _H + col])
            sl = pl.ds(j * 16, 16)
            nrm_v[sl] = esc_v[sl] * zi
            return 0
        lax.fori_loop(0, _EB2 * _H // 16, pair, 0)
        pltpu.sync_copy(nrm_v, out_hbm.at[pl.ds(base * _H, _EB2 * _H)])
        return 0

    lax.fori_loop(0, _NBLK2, blk_body, 0)


def _norm_pass(esc, ei, zinv):
    mesh = plsc.VectorSubcoreMesh(core_axis_name="c", subcore_axis_name="s",
                                  num_cores=_NC, num_subcores=_NS)
    fn = pl.kernel(
        _norm_body,
        out_type=jax.ShapeDtypeStruct((_E * _H,), jnp.float32),
        mesh=mesh,
        scratch_types=[
            pltpu.VMEM((_N * _H,), jnp.float32),
            pltpu.VMEM((_EB2 + 16,), jnp.int32),
            pltpu.VMEM((_EB2 * _H,), jnp.float32),
            pltpu.VMEM((_EB2 * _H,), jnp.float32),
        ],
        compiler_params=pltpu.CompilerParams(needs_layout_passes=False),
    )
    return fn(esc, ei, zinv)


def kernel(x, y, edge_index, Wq, bq, Wk, bk, Wv, bv, We, be, Wr, br,
           Wg, bg, ln_g, ln_b):
    ca, cb = _cols_ab()
    wcat = jnp.concatenate([Wq[:, ca], Wq[:, cb], Wk[:, ca], Wk[:, cb],
                            Wv[:, ca], Wv[:, cb], Wr], axis=1)
    bcat = jnp.concatenate([bq[ca], bq[cb], bk[ca], bk[cb], bv[ca], bv[cb],
                            br]).reshape(1, 4 * _D)
    q, kv, r = _proj(x, wcat, bcat)
    e = _eproj(y, jnp.concatenate([We[:, ca], We[:, cb]], axis=1),
               jnp.concatenate([be[ca], be[cb]]).reshape(1, _D))
    zero = jnp.zeros((_RPT, _D), jnp.float32)
    ei_flat = edge_index.reshape(-1)
    esc, opart, zpart = _edge_pass(q, kv, e, ei_flat, zero)
    zpart = zpart.reshape(_NC, _NP, _H)
    wgh = Wg[0:_D] + Wg[2 * _D:3 * _D]
    wgr = Wg[_D:2 * _D] - Wg[2 * _D:3 * _D]
    out, zinv = _finalize(opart, zpart, r, wgh, wgr, bg.reshape(1, _D),
                          ln_g.reshape(1, _D), ln_b.reshape(1, _D))
    nrm = _norm_pass(esc, ei_flat, zinv.reshape(-1))
    return out, nrm.reshape(_E, _H, 1)


# speedup vs baseline: 36.5568x; 1.0425x over previous
import functools

import jax
import jax.numpy as jnp
from jax import lax
from jax.experimental import pallas as pl
from jax.experimental.pallas import tpu as pltpu
from jax.experimental.pallas import tpu_sc as plsc

_N = 10000
_E = 320000
_D = 128
_H = 8
_DH = 16
_SCALE = 1.0 / (_DH ** 0.5)

_NC = 2
_NS = 16
_NW = _NC * _NS

_EB = 40
_EPW = _E // _NW
_NBLK = _EPW // _EB

_NP = 10240
_RPT = _NP // _NS
_ZR = _NP * _H // _D
_ZPT = _ZR // _NS

_EB2 = 400
_NBLK2 = _EPW // _EB2

_HIGH = jax.lax.Precision.DEFAULT


def _cols_ab():
    ar = jnp.arange(_D // 2)
    w = ar // _DH
    d = ar % _DH
    cols_a = 32 * w + d
    return cols_a, cols_a + _DH


def _rne16(x):
    xi = lax.bitcast_convert_type(x, jnp.int32)
    return lax.shift_right_logical(
        xi + 0x7FFF + (lax.shift_right_logical(xi, 16) & 1), 16)


def _pack(a, b):
    return _rne16(a) | lax.shift_left(_rne16(b), 16)


def _proj_body(x_ref, w_ref, b_ref, q_ref, kv_ref, r_ref):
    o = jnp.dot(x_ref[...], w_ref[...], preferred_element_type=jnp.float32,
                precision=_HIGH) + b_ref[...]
    hd = _D // 2
    q_ref[:, 0:hd] = _pack(o[:, 0:hd], o[:, hd:2 * hd])
    q_ref[:, hd:2 * hd] = jnp.zeros((o.shape[0], hd), jnp.int32)
    kv_ref[:, 0:hd] = _pack(o[:, 2 * hd:3 * hd], o[:, 3 * hd:4 * hd])
    kv_ref[:, hd:2 * hd] = _pack(o[:, 4 * hd:5 * hd], o[:, 5 * hd:6 * hd])
    r_ref[...] = o[:, 6 * hd:8 * hd]


def _proj(x, wcat, bcat):
    blk = 1000
    grid = _N // blk
    return pl.pallas_call(
        _proj_body,
        grid=(grid,),
        in_specs=[
            pl.BlockSpec((blk, _D), lambda i: (i, 0)),
            pl.BlockSpec((_D, 4 * _D), lambda i: (0, 0)),
            pl.BlockSpec((1, 4 * _D), lambda i: (0, 0)),
        ],
        out_specs=[
            pl.BlockSpec((blk, _D), lambda i: (i, 0)),
            pl.BlockSpec((blk, _D), lambda i: (i, 0)),
            pl.BlockSpec((blk, _D), lambda i: (i, 0)),
        ],
        out_shape=[
            jax.ShapeDtypeStruct((_N, _D), jnp.int32),
            jax.ShapeDtypeStruct((_N, _D), jnp.int32),
            jax.ShapeDtypeStruct((_N, _D), jnp.float32),
        ],
    )(x, wcat, bcat)


def _eproj_body(y_ref, w_ref, b_ref, e_ref):
    o = jnp.dot(y_ref[...], w_ref[...], preferred_element_type=jnp.float32,
                precision=_HIGH) + b_ref[...]
    hd = _D // 2
    e_ref[...] = _pack(o[:, 0:hd], o[:, hd:2 * hd])


def _eproj(y, we, be):
    blk = 4000
    grid = _E // blk
    return pl.pallas_call(
        _eproj_body,
        grid=(grid,),
        in_specs=[
            pl.BlockSpec((blk, _D), lambda i: (i, 0)),
            pl.BlockSpec((_D, _D), lambda i: (0, 0)),
            pl.BlockSpec((1, _D), lambda i: (0, 0)),
        ],
        out_specs=pl.BlockSpec((blk, _D // 2), lambda i: (i, 0)),
        out_shape=jax.ShapeDtypeStruct((_E, _D // 2), jnp.int32),
    )(y, we, be)


def _split_body(ei_ref, src_ref, dst_ref):
    src_ref[...] = ei_ref[0]
    dst_ref[...] = ei_ref[1]


def _split_ei(ei):
    return pl.pallas_call(
        _split_body,
        out_shape=[
            jax.ShapeDtypeStruct((_E,), jnp.int32),
            jax.ShapeDtypeStruct((_E,), jnp.int32),
        ],
    )(ei)


def _edge_body(q_hbm, kv_hbm, e_hbm, src_hbm, dst_hbm, zero_hbm,
               esc_hbm, opart_hbm, zpart_hbm,
               src_s0, dst_s0, dst_v0, src_s1, dst_s1, dst_v1,
               qr0, kvr0, er0, qr1, kvr1, er1,
               esc0, esc1, wrow, dsto, wz, zrowo,
               acc, zacc, sem_i0, sem_i1, sem_o0, sem_o1, sem_z):
    c = lax.axis_index("c")
    s = lax.axis_index("s")
    wid = c * _NS + s
    lane = lax.broadcasted_iota(jnp.int32, (16,), 0)
    msk8 = lane < 8
    nmsk8 = jnp.logical_not(msk8)
    zero16 = jnp.zeros((16,), jnp.float32)
    hi_mask = jnp.full((16,), -65536, jnp.int32)

    idx_sets = ((src_s0, dst_s0, dst_v0), (src_s1, dst_s1, dst_v1))
    in_sets = ((qr0, kvr0, er0), (qr1, kvr1, er1))
    out_sets = (esc0, esc1)
    in_sems = (sem_i0, sem_i1)
    out_sems = (sem_o0, sem_o1)

    row0 = s * _RPT
    zr0 = s * _ZPT
    pltpu.sync_copy(zero_hbm, acc.at[pl.ds(row0, _RPT)])
    pltpu.sync_copy(zero_hbm.at[pl.ds(0, _ZPT)], zacc.at[pl.ds(zr0, _ZPT)])
    plsc.subcore_barrier()

    base_t = wid * _EPW

    def stage_idx(db, b):
        src_s, dst_s, dst_v = idx_sets[db]
        base = pl.multiple_of(base_t + b * _EB, 8)
        pltpu.sync_copy(src_hbm.at[pl.ds(base, _EB)], src_s)
        pltpu.sync_copy(dst_hbm.at[pl.ds(base, _EB)], dst_s)
        pltpu.sync_copy(dst_hbm.at[pl.ds(base, _EB)],
                        dst_v.at[pl.ds(0, _EB)])

    def issue_gathers(db, b):
        src_s, dst_s, _ = idx_sets[db]
        qr, kvr, er = in_sets[db]
        base = pl.multiple_of(base_t + b * _EB, 8)
        pltpu.make_async_copy(e_hbm.at[pl.ds(base, _EB)], er,
                              in_sems[db]).start()
        pltpu.make_async_copy(q_hbm.at[dst_s], qr, in_sems[db]).start()
        pltpu.make_async_copy(kv_hbm.at[src_s], kvr, in_sems[db]).start()

    def wait_gathers(db):
        qr, kvr, er = in_sets[db]
        pltpu.make_async_copy(e_hbm.at[pl.ds(0, _EB)], er,
                              in_sems[db]).wait()
        pltpu.make_async_copy(q_hbm.at[pl.ds(0, _EB)], qr,
                              in_sems[db]).wait()
        pltpu.make_async_copy(kv_hbm.at[pl.ds(0, _EB)], kvr,
                              in_sems[db]).wait()

    def drain_outs(db):
        pltpu.make_async_copy(out_sets[db].at[pl.ds(8, _EB * _H)],
                              esc_hbm.at[pl.ds(0, _EB * _H)],
                              out_sems[db]).wait()

    def drain_scat():
        pltpu.make_async_copy(wrow, acc.at[pl.ds(0, _EB)], sem_z).wait()
        pltpu.make_async_copy(wz, zacc.at[pl.ds(0, _EB)], sem_z).wait()

    def issue_outs(db, b):
        base = pl.multiple_of(base_t + b * _EB, 8)
        pltpu.make_async_copy(out_sets[db].at[pl.ds(8, _EB * _H)],
                              esc_hbm.at[pl.ds(base * _H, _EB * _H)],
                              out_sems[db]).start()
        pltpu.make_async_copy(wrow, acc.at[dsto], sem_z).start(add=True)
        pltpu.make_async_copy(wz, zacc.at[zrowo], sem_z).start(add=True)

    def unpk(ref, i, base_col, w):
        b32 = ref[i, pl.ds(base_col + w * 16, 16)]
        lo = plsc.bitcast(lax.shift_left(b32, 16), jnp.float32)
        hi = plsc.bitcast(lax.bitwise_and(b32, hi_mask), jnp.float32)
        return lo, hi

    def compute(db):
        _, _, dst_v = idx_sets[db]
        qr, kvr, er = in_sets[db]
        esc_v = out_sets[db]

        for cc in range(3):
            dvc = dst_v[pl.ds(cc * 16, 16)]
            rem = _EB - cc * 16
            m = None if rem >= 16 else (lane < rem)
            plsc.store_scatter(dsto, [lane + cc * 16], dvc, mask=m)
            plsc.store_scatter(zrowo, [lane + cc * 16],
                               lax.shift_right_logical(dvc, 4), mask=m)

        def pair_body(p, _):
            sv = zero16
            for eo in (0, 1):
                i = 2 * p + eo
                for w in range(_H // 2):
                    q_lo, q_hi = unpk(qr, i, 0, w)
                    k_lo, k_hi = unpk(kvr, i, 0, w)
                    e_lo, e_hi = unpk(er, i, 0, w)
                    s_lo = jnp.sum(q_lo * (k_lo + e_lo))
                    s_hi = jnp.sum(q_hi * (k_hi + e_hi))
                    sv = jnp.where(lane == eo * 8 + 2 * w, s_lo, sv)
                    sv = jnp.where(lane == eo * 8 + 2 * w + 1, s_hi, sv)
            es = jnp.exp(sv)
            esc_v[pl.ds(8 + p * 16, 16)] = es
            i0 = 2 * p
            i1 = 2 * p + 1
            for eo in (0, 1):
                i = 2 * p + eo
                for w in range(_H // 2):
                    v_lo, v_hi = unpk(kvr, i, _D // 2, w)
                    e_lo, e_hi = unpk(er, i, 0, w)
                    wrow[i, pl.ds((2 * w) * _DH, _DH)] = (
                        es[eo * 8 + 2 * w] * (v_lo + e_lo))
                    wrow[i, pl.ds((2 * w + 1) * _DH, _DH)] = (
                        es[eo * 8 + 2 * w + 1] * (v_hi + e_hi))
            eva = esc_v[pl.ds(p * 16, 16)]
            evc = esc_v[pl.ds(16 + p * 16, 16)]
            dvv = dst_v[pl.ds(2 * p, 16)]
            d0 = dvv[0]
            d1 = dvv[1]
            for cz in range(_D // 16):
                wz[i0, pl.ds(cz * 16, 16)] = zero16
                wz[i1, pl.ds(cz * 16, 16)] = zero16
            c0 = (d0 & 15) >> 1
            c1 = (d1 & 15) >> 1
            val0 = jnp.where((d0 & 1) == 0, jnp.where(msk8, es, 0.0),
                             jnp.where(nmsk8, eva, 0.0))
            val1 = jnp.where((d1 & 1) == 0, jnp.where(msk8, evc, 0.0),
                             jnp.where(nmsk8, es, 0.0))
            wz[i0, pl.ds(c0 * 16, 16)] = val0
            wz[i1, pl.ds(c1 * 16, 16)] = val1
            return 0
        lax.fori_loop(0, _EB // 2, pair_body, 0)

    stage_idx(0, 0)
    issue_gathers(0, 0)

    def iter_body(g, _):
        for db in (0, 1):
            b = 2 * g + db
            nx = 1 - db

            @pl.when(b + 1 < _NBLK)
            def _():
                stage_idx(nx, b + 1)
                issue_gathers(nx, b + 1)

            wait_gathers(db)

            @pl.when(b >= 2)
            def _():
                drain_outs(db)

            @pl.when(b >= 1)
            def _():
                drain_scat()

            compute(db)
            issue_outs(db, b)
        return 0

    lax.fori_loop(0, _NBLK // 2, iter_body, 0)
    drain_outs(0)
    drain_outs(1)
    drain_scat()

    plsc.subcore_barrier()
    pltpu.sync_copy(acc.at[pl.ds(row0, _RPT)],
                    opart_hbm.at[c, pl.ds(row0, _RPT)])
    pltpu.sync_copy(zacc.at[pl.ds(zr0, _ZPT)],
                    zpart_hbm.at[c, pl.ds(zr0, _ZPT)])


def _edge_pass(q, kv, e, src, dst, zero):
    mesh = plsc.VectorSubcoreMesh(core_axis_name="c", subcore_axis_name="s",
                                  num_cores=_NC, num_subcores=_NS)
    idx_t = [pltpu.VMEM((_EB,), jnp.int32),
             pltpu.VMEM((_EB,), jnp.int32),
             pltpu.VMEM((_EB + 16,), jnp.int32)]
    in_t = [pltpu.VMEM((_EB, _D), jnp.int32),
            pltpu.VMEM((_EB, _D), jnp.int32),
            pltpu.VMEM((_EB, _D // 2), jnp.int32)]
    out_t = [pltpu.VMEM((8 + _EB * _H + 16,), jnp.float32)]
    fn = pl.kernel(
        _edge_body,
        out_type=[
            jax.ShapeDtypeStruct((_E * _H,), jnp.float32),
            jax.ShapeDtypeStruct((_NC, _NP, _D), jnp.float32),
            jax.ShapeDtypeStruct((_NC, _ZR, _D), jnp.float32),
        ],
        mesh=mesh,
        scratch_types=(idx_t + idx_t + in_t + in_t + out_t + out_t + [
            pltpu.VMEM((_EB, _D), jnp.float32),
            pltpu.VMEM((_EB,), jnp.int32),
            pltpu.VMEM((_EB, _D), jnp.float32),
            pltpu.VMEM((_EB,), jnp.int32),
            pltpu.VMEM_SHARED((_NP, _D), jnp.float32),
            pltpu.VMEM_SHARED((_ZR, _D), jnp.float32),
            pltpu.SemaphoreType.DMA,
            pltpu.SemaphoreType.DMA,
            pltpu.SemaphoreType.DMA,
            pltpu.SemaphoreType.DMA,
            pltpu.SemaphoreType.DMA,
        ]),
        compiler_params=pltpu.CompilerParams(needs_layout_passes=False),
    )
    return fn(q, kv, e, src, dst, zero)


def _fin_body(op_ref, zp_ref, r_ref, wgh_ref, wgr_ref, bg_ref, lng_ref,
              lnb_ref, out_ref):
    blk = op_ref.shape[1]
    osum = op_ref[0] + op_ref[1]
    z = zp_ref[0] + zp_ref[1]
    zinv = _SCALE / (1e-8 + z)
    h = (osum.reshape(blk, _H, _DH) * zinv[:, :, None]).reshape(blk, _D)
    r = r_ref[...]
    g = jnp.dot(h, wgh_ref[...], preferred_element_type=jnp.float32,
                precision=_HIGH)
    g += jnp.dot(r, wgr_ref[...], preferred_element_type=jnp.float32,
                 precision=_HIGH)
    b = jax.nn.sigmoid(g + bg_ref[...])
    hb = h - b * h + b * r
    mu = jnp.mean(hb, axis=1, keepdims=True)
    var = jnp.mean((hb - mu) ** 2, axis=1, keepdims=True)
    ln = (hb - mu) / jnp.sqrt(var + 1e-5) * lng_ref[...] + lnb_ref[...]
    out_ref[...] = jnp.where(ln >= 0, ln, 0.01 * ln)


def _finalize(opart, zpart, r, wgh, wgr, bg, lng, lnb):
    blk = 2000
    grid = _N // blk
    return pl.pallas_call(
        _fin_body,
        grid=(grid,),
        in_specs=[
            pl.BlockSpec((_NC, blk, _D), lambda i: (0, i, 0)),
            pl.BlockSpec((_NC, blk, _H), lambda i: (0, i, 0)),
            pl.BlockSpec((blk, _D), lambda i: (i, 0)),
            pl.BlockSpec((_D, _D), lambda i: (0, 0)),
            pl.BlockSpec((_D, _D), lambda i: (0, 0)),
            pl.BlockSpec((1, _D), lambda i: (0, 0)),
            pl.BlockSpec((1, _D), lambda i: (0, 0)),
            pl.BlockSpec((1, _D), lambda i: (0, 0)),
        ],
        out_specs=pl.BlockSpec((blk, _D), lambda i: (i, 0)),
        out_shape=jax.ShapeDtypeStruct((_N, _D), jnp.float32),
    )(opart, zpart, r, wgh, wgr, bg, lng, lnb)


def _norm_body(esc_hbm, dst_hbm, zpart_hbm, out_hbm,
               zv, zp0, zp1, dst_v, esc_v, nrm_v, zish):
    c = lax.axis_index("c")
    s = lax.axis_index("s")
    wid = c * _NS + s
    lane = lax.broadcasted_iota(jnp.int32, (16,), 0)
    msk8 = lane < 8
    lo3 = lane & 7

    pltpu.sync_copy(zpart_hbm.at[0, pl.ds(s * _ZPT, _ZPT)], zp0)
    pltpu.sync_copy(zpart_hbm.at[1, pl.ds(s * _ZPT, _ZPT)], zp1)

    def zrow_body(t, _):
        rr = t // (_D // 16)
        sl = pl.ds((t % (_D // 16)) * 16, 16)
        zp0[rr, sl] = _SCALE / (1e-8 + (zp0[rr, sl] + zp1[rr, sl]))
        return 0
    lax.fori_loop(0, _ZPT * (_D // 16), zrow_body, 0)
    pltpu.sync_copy(zp0, zish.at[pl.ds(s * _ZPT, _ZPT)])
    plsc.subcore_barrier()
    pltpu.sync_copy(zish, zv)

    base_t = wid * _EPW

    def blk_body(b, _):
        base = pl.multiple_of(base_t + b * _EB2, 8)
        pltpu.sync_copy(dst_hbm.at[pl.ds(base, _EB2)],
                        dst_v.at[pl.ds(0, _EB2)])
        pltpu.sync_copy(esc_hbm.at[pl.ds(base * _H, _EB2 * _H)], esc_v)

        def pair(j, _):
            dv = dst_v[pl.ds(2 * j, 16)]
            d0 = dv[0]
            d1 = dv[1]
            row = jnp.where(msk8, d0 >> 4, d1 >> 4)
            colb = jnp.where(msk8, (d0 & 15) * _H, (d1 & 15) * _H)
            zi = plsc.load_gather(zv, [row, colb + lo3])
            sl = pl.ds(j * 16, 16)
            nrm_v[sl] = esc_v[sl] * zi
            return 0
        lax.fori_loop(0, _EB2 * _H // 16, pair, 0)
        pltpu.sync_copy(nrm_v, out_hbm.at[pl.ds(base * _H, _EB2 * _H)])
        return 0

    lax.fori_loop(0, _NBLK2, blk_body, 0)


def _norm_pass(esc, dst, zpart):
    mesh = plsc.VectorSubcoreMesh(core_axis_name="c", subcore_axis_name="s",
                                  num_cores=_NC, num_subcores=_NS)
    fn = pl.kernel(
        _norm_body,
        out_type=jax.ShapeDtypeStruct((_E * _H,), jnp.float32),
        mesh=mesh,
        scratch_types=[
            pltpu.VMEM((_ZR, _D), jnp.float32),
            pltpu.VMEM((_ZPT, _D), jnp.float32),
            pltpu.VMEM((_ZPT, _D), jnp.float32),
            pltpu.VMEM((_EB2 + 16,), jnp.int32),
            pltpu.VMEM((_EB2 * _H,), jnp.float32),
            pltpu.VMEM((_EB2 * _H,), jnp.float32),
            pltpu.VMEM_SHARED((_ZR, _D), jnp.float32),
        ],
        compiler_params=pltpu.CompilerParams(needs_layout_passes=False),
    )
    return fn(esc, dst, zpart)


def kernel(x, y, edge_index, Wq, bq, Wk, bk, Wv, bv, We, be, Wr, br,
           Wg, bg, ln_g, ln_b):
    ca, cb = _cols_ab()
    wcat = jnp.concatenate([Wq[:, ca], Wq[:, cb], Wk[:, ca], Wk[:, cb],
                            Wv[:, ca], Wv[:, cb], Wr], axis=1)
    bcat = jnp.concatenate([bq[ca], bq[cb], bk[ca], bk[cb], bv[ca], bv[cb],
                            br]).reshape(1, 4 * _D)
    src, dst = _split_ei(edge_index)
    q, kv, r = _proj(x, wcat, bcat)
    e = _eproj(y, jnp.concatenate([We[:, ca], We[:, cb]], axis=1),
               jnp.concatenate([be[ca], be[cb]]).reshape(1, _D))
    zero = jnp.zeros((_RPT, _D), jnp.float32)
    esc, opart, zpart = _edge_pass(q, kv, e, src, dst, zero)
    wgh = Wg[0:_D] + Wg[2 * _D:3 * _D]
    wgr = Wg[_D:2 * _D] - Wg[2 * _D:3 * _D]
    out = _finalize(opart, zpart.reshape(_NC, _NP, _H), r, wgh, wgr,
                    bg.reshape(1, _D), ln_g.reshape(1, _D),
                    ln_b.reshape(1, _D))
    nrm = _norm_pass(esc, dst, zpart)
    return out, nrm.reshape(_E, _H, 1)


# speedup vs baseline: 38.1691x; 1.0441x over previous
import functools

import jax
import jax.numpy as jnp
from jax import lax
from jax.experimental import pallas as pl
from jax.experimental.pallas import tpu as pltpu
from jax.experimental.pallas import tpu_sc as plsc

_N = 10000
_E = 320000
_D = 128
_H = 8
_DH = 16
_SCALE = 1.0 / (_DH ** 0.5)

_NC = 2
_NS = 16
_NW = _NC * _NS

_EB = 40
_EPW = _E // _NW
_NBLK = _EPW // _EB

_NP = 10240
_RPT = _NP // _NS
_ZR = _NP * _H // _D
_ZPT = _ZR // _NS

_EB2 = 400
_NBLK2 = _EPW // _EB2

_HIGH = jax.lax.Precision.DEFAULT


def _cols_ab():
    ar = jnp.arange(_D // 2)
    w = ar // _DH
    d = ar % _DH
    cols_a = 32 * w + d
    return cols_a, cols_a + _DH


def _rne16(x):
    xi = lax.bitcast_convert_type(x, jnp.int32)
    return lax.shift_right_logical(
        xi + 0x7FFF + (lax.shift_right_logical(xi, 16) & 1), 16)


def _pack(a, b):
    return _rne16(a) | lax.shift_left(_rne16(b), 16)


def _proj_body(x_ref, w_ref, b_ref, q_ref, kv_ref, r_ref):
    o = jnp.dot(x_ref[...], w_ref[...], preferred_element_type=jnp.float32,
                precision=_HIGH) + b_ref[...]
    hd = _D // 2
    q_ref[:, 0:hd] = _pack(o[:, 0:hd], o[:, hd:2 * hd])
    q_ref[:, hd:2 * hd] = jnp.zeros((o.shape[0], hd), jnp.int32)
    kv_ref[:, 0:hd] = _pack(o[:, 2 * hd:3 * hd], o[:, 3 * hd:4 * hd])
    kv_ref[:, hd:2 * hd] = _pack(o[:, 4 * hd:5 * hd], o[:, 5 * hd:6 * hd])
    r_ref[...] = o[:, 6 * hd:8 * hd]


def _proj(x, wcat, bcat):
    blk = 1000
    grid = _N // blk
    return pl.pallas_call(
        _proj_body,
        grid=(grid,),
        in_specs=[
            pl.BlockSpec((blk, _D), lambda i: (i, 0)),
            pl.BlockSpec((_D, 4 * _D), lambda i: (0, 0)),
            pl.BlockSpec((1, 4 * _D), lambda i: (0, 0)),
        ],
        out_specs=[
            pl.BlockSpec((blk, _D), lambda i: (i, 0)),
            pl.BlockSpec((blk, _D), lambda i: (i, 0)),
            pl.BlockSpec((blk, _D), lambda i: (i, 0)),
        ],
        out_shape=[
            jax.ShapeDtypeStruct((_N, _D), jnp.int32),
            jax.ShapeDtypeStruct((_N, _D), jnp.int32),
            jax.ShapeDtypeStruct((_N, _D), jnp.float32),
        ],
    )(x, wcat, bcat)


def _eproj_body(y_ref, w_ref, b_ref, e_ref):
    o = jnp.dot(y_ref[...], w_ref[...], preferred_element_type=jnp.float32,
                precision=_HIGH) + b_ref[...]
    hd = _D // 2
    e_ref[...] = _pack(o[:, 0:hd], o[:, hd:2 * hd])


def _eproj(y, we, be):
    blk = 4000
    grid = _E // blk
    return pl.pallas_call(
        _eproj_body,
        grid=(grid,),
        in_specs=[
            pl.BlockSpec((blk, _D), lambda i: (i, 0)),
            pl.BlockSpec((_D, _D), lambda i: (0, 0)),
            pl.BlockSpec((1, _D), lambda i: (0, 0)),
        ],
        out_specs=pl.BlockSpec((blk, _D // 2), lambda i: (i, 0)),
        out_shape=jax.ShapeDtypeStruct((_E, _D // 2), jnp.int32),
    )(y, we, be)


def _split_body(ei_ref, src_ref, dst_ref):
    src_ref[...] = ei_ref[0]
    dst_ref[...] = ei_ref[1]


def _split_ei(ei):
    return pl.pallas_call(
        _split_body,
        out_shape=[
            jax.ShapeDtypeStruct((_E,), jnp.int32),
            jax.ShapeDtypeStruct((_E,), jnp.int32),
        ],
    )(ei)


def _edge_body(q_hbm, kv_hbm, e_hbm, src_hbm, dst_hbm, zero_hbm,
               esc_hbm, opart_hbm, zpart_hbm,
               src_s0, dst_s0, dst_v0, src_s1, dst_s1, dst_v1,
               qr0, kvr0, er0, qr1, kvr1, er1,
               esc0, esc1, wrow, dsto, wz, zrowo,
               acc, zacc, sem_i0, sem_i1, sem_o0, sem_o1, sem_z):
    c = lax.axis_index("c")
    s = lax.axis_index("s")
    wid = c * _NS + s
    lane = lax.broadcasted_iota(jnp.int32, (16,), 0)
    msk8 = lane < 8
    nmsk8 = jnp.logical_not(msk8)
    zero16 = jnp.zeros((16,), jnp.float32)
    hi_mask = jnp.full((16,), -65536, jnp.int32)

    idx_sets = ((src_s0, dst_s0, dst_v0), (src_s1, dst_s1, dst_v1))
    in_sets = ((qr0, kvr0, er0), (qr1, kvr1, er1))
    out_sets = (esc0, esc1)
    in_sems = (sem_i0, sem_i1)
    out_sems = (sem_o0, sem_o1)

    row0 = s * _RPT
    zr0 = s * _ZPT
    pltpu.sync_copy(zero_hbm, acc.at[pl.ds(row0, _RPT)])
    pltpu.sync_copy(zero_hbm.at[pl.ds(0, _ZPT)], zacc.at[pl.ds(zr0, _ZPT)])
    plsc.subcore_barrier()

    base_t = wid * _EPW

    def stage_idx(db, b):
        src_s, dst_s, dst_v = idx_sets[db]
        base = pl.multiple_of(base_t + b * _EB, 8)
        pltpu.sync_copy(src_hbm.at[pl.ds(base, _EB)], src_s)
        pltpu.sync_copy(dst_hbm.at[pl.ds(base, _EB)], dst_s)
        pltpu.sync_copy(dst_hbm.at[pl.ds(base, _EB)],
                        dst_v.at[pl.ds(0, _EB)])

    def issue_gathers(db, b):
        src_s, dst_s, _ = idx_sets[db]
        qr, kvr, er = in_sets[db]
        base = pl.multiple_of(base_t + b * _EB, 8)
        pltpu.make_async_copy(e_hbm.at[pl.ds(base, _EB)], er,
                              in_sems[db]).start()
        pltpu.make_async_copy(q_hbm.at[dst_s], qr, in_sems[db]).start()
        pltpu.make_async_copy(kv_hbm.at[src_s], kvr, in_sems[db]).start()

    def wait_gathers(db):
        qr, kvr, er = in_sets[db]
        pltpu.make_async_copy(e_hbm.at[pl.ds(0, _EB)], er,
                              in_sems[db]).wait()
        pltpu.make_async_copy(q_hbm.at[pl.ds(0, _EB)], qr,
                              in_sems[db]).wait()
        pltpu.make_async_copy(kv_hbm.at[pl.ds(0, _EB)], kvr,
                              in_sems[db]).wait()

    def drain_outs(db):
        pltpu.make_async_copy(out_sets[db].at[pl.ds(8, _EB * _H)],
                              esc_hbm.at[pl.ds(0, _EB * _H)],
                              out_sems[db]).wait()

    def drain_scat():
        pltpu.make_async_copy(wrow, acc.at[pl.ds(0, _EB)], sem_z).wait()
        pltpu.make_async_copy(wz, zacc.at[pl.ds(0, _EB)], sem_z).wait()

    def issue_outs(db, b):
        base = pl.multiple_of(base_t + b * _EB, 8)
        pltpu.make_async_copy(out_sets[db].at[pl.ds(8, _EB * _H)],
                              esc_hbm.at[pl.ds(base * _H, _EB * _H)],
                              out_sems[db]).start()
        pltpu.make_async_copy(wrow, acc.at[dsto], sem_z).start(add=True)
        pltpu.make_async_copy(wz, zacc.at[zrowo], sem_z).start(add=True)

    def unpk32(b32):
        lo = plsc.bitcast(lax.shift_left(b32, 16), jnp.float32)
        hi = plsc.bitcast(lax.bitwise_and(b32, hi_mask), jnp.float32)
        return lo, hi

    def compute(db):
        _, _, dst_v = idx_sets[db]
        qr, kvr, er = in_sets[db]
        esc_v = out_sets[db]

        for cc in range(3):
            dvc = dst_v[pl.ds(cc * 16, 16)]
            rem = _EB - cc * 16
            m = None if rem >= 16 else (lane < rem)
            plsc.store_scatter(dsto, [lane + cc * 16], dvc, mask=m)
            plsc.store_scatter(zrowo, [lane + cc * 16],
                               lax.shift_right_logical(dvc, 4), mask=m)

        def pair_body(p, _):
            sv = zero16
            ebf = []
            for eo in (0, 1):
                i = 2 * p + eo
                for w in range(_H // 2):
                    sl = pl.ds(w * 16, 16)
                    eb = plsc.bitcast(er[i, sl], jnp.bfloat16)
                    kb = plsc.bitcast(kvr[i, sl], jnp.bfloat16)
                    ebf.append(eb)
                    ke_lo, ke_hi = unpk32(plsc.bitcast(kb + eb, jnp.int32))
                    q_lo, q_hi = unpk32(qr[i, sl])
                    s_lo = jnp.sum(q_lo * ke_lo)
                    s_hi = jnp.sum(q_hi * ke_hi)
                    sv = jnp.where(lane == eo * 8 + 2 * w, s_lo, sv)
                    sv = jnp.where(lane == eo * 8 + 2 * w + 1, s_hi, sv)
            es = jnp.exp(sv)
            esc_v[pl.ds(8 + p * 16, 16)] = es
            i0 = 2 * p
            i1 = 2 * p + 1
            for eo in (0, 1):
                i = 2 * p + eo
                for w in range(_H // 2):
                    vb = plsc.bitcast(kvr[i, pl.ds(_D // 2 + w * 16, 16)],
                                      jnp.bfloat16)
                    ve_lo, ve_hi = unpk32(
                        plsc.bitcast(vb + ebf[eo * 4 + w], jnp.int32))
                    wrow[i, pl.ds((2 * w) * _DH, _DH)] = (
                        es[eo * 8 + 2 * w] * ve_lo)
                    wrow[i, pl.ds((2 * w + 1) * _DH, _DH)] = (
                        es[eo * 8 + 2 * w + 1] * ve_hi)
            eva = esc_v[pl.ds(p * 16, 16)]
            evc = esc_v[pl.ds(16 + p * 16, 16)]
            dvv = dst_v[pl.ds(2 * p, 16)]
            d0 = dvv[0]
            d1 = dvv[1]
            for cz in range(_D // 16):
                wz[i0, pl.ds(cz * 16, 16)] = zero16
                wz[i1, pl.ds(cz * 16, 16)] = zero16
            c0 = (d0 & 15) >> 1
            c1 = (d1 & 15) >> 1
            val0 = jnp.where((d0 & 1) == 0, jnp.where(msk8, es, 0.0),
                             jnp.where(nmsk8, eva, 0.0))
            val1 = jnp.where((d1 & 1) == 0, jnp.where(msk8, evc, 0.0),
                             jnp.where(nmsk8, es, 0.0))
            wz[i0, pl.ds(c0 * 16, 16)] = val0
            wz[i1, pl.ds(c1 * 16, 16)] = val1
            return 0
        lax.fori_loop(0, _EB // 2, pair_body, 0)

    stage_idx(0, 0)
    issue_gathers(0, 0)

    def iter_body(g, _):
        for db in (0, 1):
            b = 2 * g + db
            nx = 1 - db

            @pl.when(b + 1 < _NBLK)
            def _():
                stage_idx(nx, b + 1)
                issue_gathers(nx, b + 1)

            wait_gathers(db)

            @pl.when(b >= 2)
            def _():
                drain_outs(db)

            @pl.when(b >= 1)
            def _():
                drain_scat()

            compute(db)
            issue_outs(db, b)
        return 0

    lax.fori_loop(0, _NBLK // 2, iter_body, 0)
    drain_outs(0)
    drain_outs(1)
    drain_scat()

    plsc.subcore_barrier()
    pltpu.sync_copy(acc.at[pl.ds(row0, _RPT)],
                    opart_hbm.at[c, pl.ds(row0, _RPT)])
    pltpu.sync_copy(zacc.at[pl.ds(zr0, _ZPT)],
                    zpart_hbm.at[c, pl.ds(zr0, _ZPT)])


def _edge_pass(q, kv, e, src, dst, zero):
    mesh = plsc.VectorSubcoreMesh(core_axis_name="c", subcore_axis_name="s",
                                  num_cores=_NC, num_subcores=_NS)
    idx_t = [pltpu.VMEM((_EB,), jnp.int32),
             pltpu.VMEM((_EB,), jnp.int32),
             pltpu.VMEM((_EB + 16,), jnp.int32)]
    in_t = [pltpu.VMEM((_EB, _D), jnp.int32),
            pltpu.VMEM((_EB, _D), jnp.int32),
            pltpu.VMEM((_EB, _D // 2), jnp.int32)]
    out_t = [pltpu.VMEM((8 + _EB * _H + 16,), jnp.float32)]
    fn = pl.kernel(
        _edge_body,
        out_type=[
            jax.ShapeDtypeStruct((_E * _H,), jnp.float32),
            jax.ShapeDtypeStruct((_NC, _NP, _D), jnp.float32),
            jax.ShapeDtypeStruct((_NC, _ZR, _D), jnp.float32),
        ],
        mesh=mesh,
        scratch_types=(idx_t + idx_t + in_t + in_t + out_t + out_t + [
            pltpu.VMEM((_EB, _D), jnp.float32),
            pltpu.VMEM((_EB,), jnp.int32),
            pltpu.VMEM((_EB, _D), jnp.float32),
            pltpu.VMEM((_EB,), jnp.int32),
            pltpu.VMEM_SHARED((_NP, _D), jnp.float32),
            pltpu.VMEM_SHARED((_ZR, _D), jnp.float32),
            pltpu.SemaphoreType.DMA,
            pltpu.SemaphoreType.DMA,
            pltpu.SemaphoreType.DMA,
            pltpu.SemaphoreType.DMA,
            pltpu.SemaphoreType.DMA,
        ]),
        compiler_params=pltpu.CompilerParams(needs_layout_passes=False),
    )
    return fn(q, kv, e, src, dst, zero)


def _fin_body(op_ref, zp_ref, r_ref, wgh_ref, wgr_ref, bg_ref, lng_ref,
              lnb_ref, out_ref):
    blk = op_ref.shape[1]
    osum = op_ref[0] + op_ref[1]
    z = zp_ref[0] + zp_ref[1]
    zinv = _SCALE / (1e-8 + z)
    h = (osum.reshape(blk, _H, _DH) * zinv[:, :, None]).reshape(blk, _D)
    r = r_ref[...]
    g = jnp.dot(h, wgh_ref[...], preferred_element_type=jnp.float32,
                precision=_HIGH)
    g += jnp.dot(r, wgr_ref[...], preferred_element_type=jnp.float32,
                 precision=_HIGH)
    b = jax.nn.sigmoid(g + bg_ref[...])
    hb = h - b * h + b * r
    mu = jnp.mean(hb, axis=1, keepdims=True)
    var = jnp.mean((hb - mu) ** 2, axis=1, keepdims=True)
    ln = (hb - mu) / jnp.sqrt(var + 1e-5) * lng_ref[...] + lnb_ref[...]
    out_ref[...] = jnp.where(ln >= 0, ln, 0.01 * ln)


def _finalize(opart, zpart, r, wgh, wgr, bg, lng, lnb):
    blk = 2000
    grid = _N // blk
    return pl.pallas_call(
        _fin_body,
        grid=(grid,),
        in_specs=[
            pl.BlockSpec((_NC, blk, _D), lambda i: (0, i, 0)),
            pl.BlockSpec((_NC, blk, _H), lambda i: (0, i, 0)),
            pl.BlockSpec((blk, _D), lambda i: (i, 0)),
            pl.BlockSpec((_D, _D), lambda i: (0, 0)),
            pl.BlockSpec((_D, _D), lambda i: (0, 0)),
            pl.BlockSpec((1, _D), lambda i: (0, 0)),
            pl.BlockSpec((1, _D), lambda i: (0, 0)),
            pl.BlockSpec((1, _D), lambda i: (0, 0)),
        ],
        out_specs=pl.BlockSpec((blk, _D), lambda i: (i, 0)),
        out_shape=jax.ShapeDtypeStruct((_N, _D), jnp.float32),
    )(opart, zpart, r, wgh, wgr, bg, lng, lnb)


def _norm_body(esc_hbm, dst_hbm, zpart_hbm, out_hbm,
               zv, zp0, zp1, dst_v, esc_v, nrm_v, zish):
    c = lax.axis_index("c")
    s = lax.axis_index("s")
    wid = c * _NS + s
    lane = lax.broadcasted_iota(jnp.int32, (16,), 0)
    msk8 = lane < 8
    lo3 = lane & 7

    pltpu.sync_copy(zpart_hbm.at[0, pl.ds(s * _ZPT, _ZPT)], zp0)
    pltpu.sync_copy(zpart_hbm.at[1, pl.ds(s * _ZPT, _ZPT)], zp1)

    def zrow_body(t, _):
        rr = t // (_D // 16)
        sl = pl.ds((t % (_D // 16)) * 16, 16)
        zp0[rr, sl] = _SCALE / (1e-8 + (zp0[rr, sl] + zp1[rr, sl]))
        return 0
    lax.fori_loop(0, _ZPT * (_D // 16), zrow_body, 0)
    pltpu.sync_copy(zp0, zish.at[pl.ds(s * _ZPT, _ZPT)])
    plsc.subcore_barrier()
    pltpu.sync_copy(zish, zv)

    base_t = wid * _EPW

    def blk_body(b, _):
        base = pl.multiple_of(base_t + b * _EB2, 8)
        pltpu.sync_copy(dst_hbm.at[pl.ds(base, _EB2)],
                        dst_v.at[pl.ds(0, _EB2)])
        pltpu.sync_copy(esc_hbm.at[pl.ds(base * _H, _EB2 * _H)], esc_v)

        def pair(j, _):
            dv = dst_v[pl.ds(2 * j, 16)]
            d0 = dv[0]
            d1 = dv[1]
            row = jnp.where(msk8, d0 >> 4, d1 >> 4)
            colb = jnp.where(msk8, (d0 & 15) * _H, (d1 & 15) * _H)
            zi = plsc.load_gather(zv, [row, colb + lo3])
            sl = pl.ds(j * 16, 16)
            nrm_v[sl] = esc_v[sl] * zi
            return 0
        lax.fori_loop(0, _EB2 * _H // 16, pair, 0)
        pltpu.sync_copy(nrm_v, out_hbm.at[pl.ds(base * _H, _EB2 * _H)])
        return 0

    lax.fori_loop(0, _NBLK2, blk_body, 0)


def _norm_pass(esc, dst, zpart):
    mesh = plsc.VectorSubcoreMesh(core_axis_name="c", subcore_axis_name="s",
                                  num_cores=_NC, num_subcores=_NS)
    fn = pl.kernel(
        _norm_body,
        out_type=jax.ShapeDtypeStruct((_E * _H,), jnp.float32),
        mesh=mesh,
        scratch_types=[
            pltpu.VMEM((_ZR, _D), jnp.float32),
            pltpu.VMEM((_ZPT, _D), jnp.float32),
            pltpu.VMEM((_ZPT, _D), jnp.float32),
            pltpu.VMEM((_EB2 + 16,), jnp.int32),
            pltpu.VMEM((_EB2 * _H,), jnp.float32),
            pltpu.VMEM((_EB2 * _H,), jnp.float32),
            pltpu.VMEM_SHARED((_ZR, _D), jnp.float32),
        ],
        compiler_params=pltpu.CompilerParams(needs_layout_passes=False),
    )
    return fn(esc, dst, zpart)


def kernel(x, y, edge_index, Wq, bq, Wk, bk, Wv, bv, We, be, Wr, br,
           Wg, bg, ln_g, ln_b):
    ca, cb = _cols_ab()
    wcat = jnp.concatenate([Wq[:, ca], Wq[:, cb], Wk[:, ca], Wk[:, cb],
                            Wv[:, ca], Wv[:, cb], Wr], axis=1)
    bcat = jnp.concatenate([bq[ca], bq[cb], bk[ca], bk[cb], bv[ca], bv[cb],
                            br]).reshape(1, 4 * _D)
    src, dst = _split_ei(edge_index)
    q, kv, r = _proj(x, wcat, bcat)
    e = _eproj(y, jnp.concatenate([We[:, ca], We[:, cb]], axis=1),
               jnp.concatenate([be[ca], be[cb]]).reshape(1, _D))
    zero = jnp.zeros((_RPT, _D), jnp.float32)
    esc, opart, zpart = _edge_pass(q, kv, e, src, dst, zero)
    wgh = Wg[0:_D] + Wg[2 * _D:3 * _D]
    wgr = Wg[_D:2 * _D] - Wg[2 * _D:3 * _D]
    out = _finalize(opart, zpart.reshape(_NC, _NP, _H), r, wgh, wgr,
                    bg.reshape(1, _D), ln_g.reshape(1, _D),
                    ln_b.reshape(1, _D))
    nrm = _norm_pass(esc, dst, zpart)
    return out, nrm.reshape(_E, _H, 1)


# speedup vs baseline: 39.3047x; 1.0298x over previous
import functools

import jax
import jax.numpy as jnp
from jax import lax
from jax.experimental import pallas as pl
from jax.experimental.pallas import tpu as pltpu
from jax.experimental.pallas import tpu_sc as plsc

_N = 10000
_E = 320000
_D = 128
_H = 8
_DH = 16
_SCALE = 1.0 / (_DH ** 0.5)

_NC = 2
_NS = 16
_NW = _NC * _NS

_EB = 40
_EPW = _E // _NW
_NBLK = _EPW // _EB

_NP = 10240
_RPT = _NP // _NS
_ZR = _NP * _H // _D
_ZPT = _ZR // _NS

_EB2 = 200
_NBLK2 = _EPW // _EB2

_HIGH = jax.lax.Precision.DEFAULT


def _cols_ab():
    ar = jnp.arange(_D // 2)
    w = ar // _DH
    d = ar % _DH
    cols_a = 32 * w + d
    return cols_a, cols_a + _DH


def _rne16(x):
    xi = lax.bitcast_convert_type(x, jnp.int32)
    return lax.shift_right_logical(
        xi + 0x7FFF + (lax.shift_right_logical(xi, 16) & 1), 16)


def _pack(a, b):
    return _rne16(a) | lax.shift_left(_rne16(b), 16)


def _proj_body(x_ref, w_ref, b_ref, q_ref, kv_ref, r_ref):
    o = jnp.dot(x_ref[...], w_ref[...], preferred_element_type=jnp.float32,
                precision=_HIGH) + b_ref[...]
    hd = _D // 2
    q_ref[:, 0:hd] = _pack(o[:, 0:hd], o[:, hd:2 * hd])
    q_ref[:, hd:2 * hd] = jnp.zeros((o.shape[0], hd), jnp.int32)
    kv_ref[:, 0:hd] = _pack(o[:, 2 * hd:3 * hd], o[:, 3 * hd:4 * hd])
    kv_ref[:, hd:2 * hd] = _pack(o[:, 4 * hd:5 * hd], o[:, 5 * hd:6 * hd])
    r_ref[...] = o[:, 6 * hd:8 * hd]


def _proj(x, wcat, bcat):
    blk = 1000
    grid = _N // blk
    return pl.pallas_call(
        _proj_body,
        grid=(grid,),
        in_specs=[
            pl.BlockSpec((blk, _D), lambda i: (i, 0)),
            pl.BlockSpec((_D, 4 * _D), lambda i: (0, 0)),
            pl.BlockSpec((1, 4 * _D), lambda i: (0, 0)),
        ],
        out_specs=[
            pl.BlockSpec((blk, _D), lambda i: (i, 0)),
            pl.BlockSpec((blk, _D), lambda i: (i, 0)),
            pl.BlockSpec((blk, _D), lambda i: (i, 0)),
        ],
        out_shape=[
            jax.ShapeDtypeStruct((_N, _D), jnp.int32),
            jax.ShapeDtypeStruct((_N, _D), jnp.int32),
            jax.ShapeDtypeStruct((_N, _D), jnp.float32),
        ],
    )(x, wcat, bcat)


def _eproj_body(y_ref, w_ref, b_ref, e_ref):
    o = jnp.dot(y_ref[...], w_ref[...], preferred_element_type=jnp.float32,
                precision=_HIGH) + b_ref[...]
    hd = _D // 2
    e_ref[...] = _pack(o[:, 0:hd], o[:, hd:2 * hd])


def _eproj(y, we, be):
    blk = 4000
    grid = _E // blk
    return pl.pallas_call(
        _eproj_body,
        grid=(grid,),
        in_specs=[
            pl.BlockSpec((blk, _D), lambda i: (i, 0)),
            pl.BlockSpec((_D, _D), lambda i: (0, 0)),
            pl.BlockSpec((1, _D), lambda i: (0, 0)),
        ],
        out_specs=pl.BlockSpec((blk, _D // 2), lambda i: (i, 0)),
        out_shape=jax.ShapeDtypeStruct((_E, _D // 2), jnp.int32),
    )(y, we, be)


def _split_body(ei_ref, src_ref, dst_ref):
    src_ref[...] = ei_ref[0]
    dst_ref[...] = ei_ref[1]


def _split_ei(ei):
    return pl.pallas_call(
        _split_body,
        out_shape=[
            jax.ShapeDtypeStruct((_E,), jnp.int32),
            jax.ShapeDtypeStruct((_E,), jnp.int32),
        ],
    )(ei)


def _edge_body(q_hbm, kv_hbm, e_hbm, src_hbm, dst_hbm, zero_hbm,
               esc_hbm, opart_hbm, zpart_hbm,
               src_s0, dst_s0, dst_v0, src_s1, dst_s1, dst_v1,
               qr0, kvr0, er0, qr1, kvr1, er1,
               esc0, esc1, wrow, dsto, wz, zrowo,
               acc, zacc, sem_i0, sem_i1, sem_o0, sem_o1, sem_z):
    c = lax.axis_index("c")
    s = lax.axis_index("s")
    wid = c * _NS + s
    lane = lax.broadcasted_iota(jnp.int32, (16,), 0)
    msk8 = lane < 8
    nmsk8 = jnp.logical_not(msk8)
    zero16 = jnp.zeros((16,), jnp.float32)
    hi_mask = jnp.full((16,), -65536, jnp.int32)

    idx_sets = ((src_s0, dst_s0, dst_v0), (src_s1, dst_s1, dst_v1))
    in_sets = ((qr0, kvr0, er0), (qr1, kvr1, er1))
    out_sets = (esc0, esc1)
    in_sems = (sem_i0, sem_i1)
    out_sems = (sem_o0, sem_o1)

    row0 = s * _RPT
    zr0 = s * _ZPT
    pltpu.sync_copy(zero_hbm, acc.at[pl.ds(row0, _RPT)])
    pltpu.sync_copy(zero_hbm.at[pl.ds(0, _ZPT)], zacc.at[pl.ds(zr0, _ZPT)])
    plsc.subcore_barrier()

    base_t = wid * _EPW

    def stage_idx(db, b):
        src_s, dst_s, dst_v = idx_sets[db]
        base = pl.multiple_of(base_t + b * _EB, 8)
        pltpu.sync_copy(src_hbm.at[pl.ds(base, _EB)], src_s)
        pltpu.sync_copy(dst_hbm.at[pl.ds(base, _EB)], dst_s)
        pltpu.sync_copy(dst_hbm.at[pl.ds(base, _EB)],
                        dst_v.at[pl.ds(0, _EB)])

    def issue_gathers(db, b):
        src_s, dst_s, _ = idx_sets[db]
        qr, kvr, er = in_sets[db]
        base = pl.multiple_of(base_t + b * _EB, 8)
        pltpu.make_async_copy(e_hbm.at[pl.ds(base, _EB)], er,
                              in_sems[db]).start()
        pltpu.make_async_copy(q_hbm.at[dst_s], qr, in_sems[db]).start()
        pltpu.make_async_copy(kv_hbm.at[src_s], kvr, in_sems[db]).start()

    def wait_gathers(db):
        qr, kvr, er = in_sets[db]
        pltpu.make_async_copy(e_hbm.at[pl.ds(0, _EB)], er,
                              in_sems[db]).wait()
        pltpu.make_async_copy(q_hbm.at[pl.ds(0, _EB)], qr,
                              in_sems[db]).wait()
        pltpu.make_async_copy(kv_hbm.at[pl.ds(0, _EB)], kvr,
                              in_sems[db]).wait()

    def drain_outs(db):
        pltpu.make_async_copy(out_sets[db].at[pl.ds(8, _EB * _H)],
                              esc_hbm.at[pl.ds(0, _EB * _H)],
                              out_sems[db]).wait()

    def drain_scat():
        pltpu.make_async_copy(wrow, acc.at[pl.ds(0, _EB)], sem_z).wait()
        pltpu.make_async_copy(wz, zacc.at[pl.ds(0, _EB)], sem_z).wait()

    def issue_outs(db, b):
        base = pl.multiple_of(base_t + b * _EB, 8)
        pltpu.make_async_copy(out_sets[db].at[pl.ds(8, _EB * _H)],
                              esc_hbm.at[pl.ds(base * _H, _EB * _H)],
                              out_sems[db]).start()
        pltpu.make_async_copy(wrow, acc.at[dsto], sem_z).start(add=True)
        pltpu.make_async_copy(wz, zacc.at[zrowo], sem_z).start(add=True)

    def unpk32(b32):
        lo = plsc.bitcast(lax.shift_left(b32, 16), jnp.float32)
        hi = plsc.bitcast(lax.bitwise_and(b32, hi_mask), jnp.float32)
        return lo, hi

    def compute(db):
        _, _, dst_v = idx_sets[db]
        qr, kvr, er = in_sets[db]
        esc_v = out_sets[db]

        for cc in range(3):
            dvc = dst_v[pl.ds(cc * 16, 16)]
            rem = _EB - cc * 16
            m = None if rem >= 16 else (lane < rem)
            plsc.store_scatter(dsto, [lane + cc * 16], dvc, mask=m)
            plsc.store_scatter(zrowo, [lane + cc * 16],
                               lax.shift_right_logical(dvc, 4), mask=m)

        def pair_body(p, _):
            sv = zero16
            ebf = []
            for eo in (0, 1):
                i = 2 * p + eo
                for w in range(_H // 2):
                    sl = pl.ds(w * 16, 16)
                    eb = plsc.bitcast(er[i, sl], jnp.bfloat16)
                    kb = plsc.bitcast(kvr[i, sl], jnp.bfloat16)
                    ebf.append(eb)
                    ke_lo, ke_hi = unpk32(plsc.bitcast(kb + eb, jnp.int32))
                    q_lo, q_hi = unpk32(qr[i, sl])
                    s_lo = jnp.sum(q_lo * ke_lo)
                    s_hi = jnp.sum(q_hi * ke_hi)
                    sv = jnp.where(lane == eo * 8 + 2 * w, s_lo, sv)
                    sv = jnp.where(lane == eo * 8 + 2 * w + 1, s_hi, sv)
            es = jnp.exp(sv)
            esc_v[pl.ds(8 + p * 16, 16)] = es
            i0 = 2 * p
            i1 = 2 * p + 1
            for eo in (0, 1):
                i = 2 * p + eo
                for w in range(_H // 2):
                    vb = plsc.bitcast(kvr[i, pl.ds(_D // 2 + w * 16, 16)],
                                      jnp.bfloat16)
                    ve_lo, ve_hi = unpk32(
                        plsc.bitcast(vb + ebf[eo * 4 + w], jnp.int32))
                    wrow[i, pl.ds((2 * w) * _DH, _DH)] = (
                        es[eo * 8 + 2 * w] * ve_lo)
                    wrow[i, pl.ds((2 * w + 1) * _DH, _DH)] = (
                        es[eo * 8 + 2 * w + 1] * ve_hi)
            eva = esc_v[pl.ds(p * 16, 16)]
            evc = esc_v[pl.ds(16 + p * 16, 16)]
            dvv = dst_v[pl.ds(2 * p, 16)]
            d0 = dvv[0]
            d1 = dvv[1]
            for cz in range(_D // 16):
                wz[i0, pl.ds(cz * 16, 16)] = zero16
                wz[i1, pl.ds(cz * 16, 16)] = zero16
            c0 = (d0 & 15) >> 1
            c1 = (d1 & 15) >> 1
            val0 = jnp.where((d0 & 1) == 0, jnp.where(msk8, es, 0.0),
                             jnp.where(nmsk8, eva, 0.0))
            val1 = jnp.where((d1 & 1) == 0, jnp.where(msk8, evc, 0.0),
                             jnp.where(nmsk8, es, 0.0))
            wz[i0, pl.ds(c0 * 16, 16)] = val0
            wz[i1, pl.ds(c1 * 16, 16)] = val1
            return 0
        lax.fori_loop(0, _EB // 2, pair_body, 0)

    stage_idx(0, 0)
    issue_gathers(0, 0)

    def iter_body(g, _):
        for db in (0, 1):
            b = 2 * g + db
            nx = 1 - db

            @pl.when(b + 1 < _NBLK)
            def _():
                stage_idx(nx, b + 1)
                issue_gathers(nx, b + 1)

            wait_gathers(db)

            @pl.when(b >= 2)
            def _():
                drain_outs(db)

            @pl.when(b >= 1)
            def _():
                drain_scat()

            compute(db)
            issue_outs(db, b)
        return 0

    lax.fori_loop(0, _NBLK // 2, iter_body, 0)
    drain_outs(0)
    drain_outs(1)
    drain_scat()

    plsc.subcore_barrier()
    pltpu.sync_copy(acc.at[pl.ds(row0, _RPT)],
                    opart_hbm.at[c, pl.ds(row0, _RPT)])
    pltpu.sync_copy(zacc.at[pl.ds(zr0, _ZPT)],
                    zpart_hbm.at[c, pl.ds(zr0, _ZPT)])


def _edge_pass(q, kv, e, src, dst, zero):
    mesh = plsc.VectorSubcoreMesh(core_axis_name="c", subcore_axis_name="s",
                                  num_cores=_NC, num_subcores=_NS)
    idx_t = [pltpu.VMEM((_EB,), jnp.int32),
             pltpu.VMEM((_EB,), jnp.int32),
             pltpu.VMEM((_EB + 16,), jnp.int32)]
    in_t = [pltpu.VMEM((_EB, _D), jnp.int32),
            pltpu.VMEM((_EB, _D), jnp.int32),
            pltpu.VMEM((_EB, _D // 2), jnp.int32)]
    out_t = [pltpu.VMEM((8 + _EB * _H + 16,), jnp.float32)]
    fn = pl.kernel(
        _edge_body,
        out_type=[
            jax.ShapeDtypeStruct((_E * _H,), jnp.float32),
            jax.ShapeDtypeStruct((_NC, _NP, _D), jnp.float32),
            jax.ShapeDtypeStruct((_NC, _ZR, _D), jnp.float32),
        ],
        mesh=mesh,
        scratch_types=(idx_t + idx_t + in_t + in_t + out_t + out_t + [
            pltpu.VMEM((_EB, _D), jnp.float32),
            pltpu.VMEM((_EB,), jnp.int32),
            pltpu.VMEM((_EB, _D), jnp.float32),
            pltpu.VMEM((_EB,), jnp.int32),
            pltpu.VMEM_SHARED((_NP, _D), jnp.float32),
            pltpu.VMEM_SHARED((_ZR, _D), jnp.float32),
            pltpu.SemaphoreType.DMA,
            pltpu.SemaphoreType.DMA,
            pltpu.SemaphoreType.DMA,
            pltpu.SemaphoreType.DMA,
            pltpu.SemaphoreType.DMA,
        ]),
        compiler_params=pltpu.CompilerParams(needs_layout_passes=False),
    )
    return fn(q, kv, e, src, dst, zero)


def _fin_body(op_ref, zp_ref, r_ref, wgh_ref, wgr_ref, bg_ref, lng_ref,
              lnb_ref, out_ref):
    blk = op_ref.shape[1]
    osum = op_ref[0] + op_ref[1]
    z = zp_ref[0] + zp_ref[1]
    zinv = _SCALE / (1e-8 + z)
    h = (osum.reshape(blk, _H, _DH) * zinv[:, :, None]).reshape(blk, _D)
    r = r_ref[...]
    g = jnp.dot(h, wgh_ref[...], preferred_element_type=jnp.float32,
                precision=_HIGH)
    g += jnp.dot(r, wgr_ref[...], preferred_element_type=jnp.float32,
                 precision=_HIGH)
    b = jax.nn.sigmoid(g + bg_ref[...])
    hb = h - b * h + b * r
    mu = jnp.mean(hb, axis=1, keepdims=True)
    var = jnp.mean((hb - mu) ** 2, axis=1, keepdims=True)
    ln = (hb - mu) / jnp.sqrt(var + 1e-5) * lng_ref[...] + lnb_ref[...]
    out_ref[...] = jnp.where(ln >= 0, ln, 0.01 * ln)


def _finalize(opart, zpart, r, wgh, wgr, bg, lng, lnb):
    blk = 2000
    grid = _N // blk
    return pl.pallas_call(
        _fin_body,
        grid=(grid,),
        in_specs=[
            pl.BlockSpec((_NC, blk, _D), lambda i: (0, i, 0)),
            pl.BlockSpec((_NC, blk, _H), lambda i: (0, i, 0)),
            pl.BlockSpec((blk, _D), lambda i: (i, 0)),
            pl.BlockSpec((_D, _D), lambda i: (0, 0)),
            pl.BlockSpec((_D, _D), lambda i: (0, 0)),
            pl.BlockSpec((1, _D), lambda i: (0, 0)),
            pl.BlockSpec((1, _D), lambda i: (0, 0)),
            pl.BlockSpec((1, _D), lambda i: (0, 0)),
        ],
        out_specs=pl.BlockSpec((blk, _D), lambda i: (i, 0)),
        out_shape=jax.ShapeDtypeStruct((_N, _D), jnp.float32),
    )(opart, zpart, r, wgh, wgr, bg, lng, lnb)


def _norm_body(esc_hbm, dst_hbm, zpart_hbm, out_hbm,
               zv, zp0, zp1, dst_v0, dst_v1, esc_v0, esc_v1, nrm_v0, nrm_v1,
               zish, sem_i0, sem_i1, sem_o0, sem_o1):
    c = lax.axis_index("c")
    s = lax.axis_index("s")
    wid = c * _NS + s
    lane = lax.broadcasted_iota(jnp.int32, (16,), 0)
    msk8 = lane < 8
    lo3 = lane & 7

    pltpu.sync_copy(zpart_hbm.at[0, pl.ds(s * _ZPT, _ZPT)], zp0)
    pltpu.sync_copy(zpart_hbm.at[1, pl.ds(s * _ZPT, _ZPT)], zp1)

    def zrow_body(t, _):
        rr = t // (_D // 16)
        sl = pl.ds((t % (_D // 16)) * 16, 16)
        zp0[rr, sl] = _SCALE / (1e-8 + (zp0[rr, sl] + zp1[rr, sl]))
        return 0
    lax.fori_loop(0, _ZPT * (_D // 16), zrow_body, 0)
    pltpu.sync_copy(zp0, zish.at[pl.ds(s * _ZPT, _ZPT)])
    plsc.subcore_barrier()
    pltpu.sync_copy(zish, zv)

    base_t = wid * _EPW

    dsts = (dst_v0, dst_v1)
    escs = (esc_v0, esc_v1)
    nrms = (nrm_v0, nrm_v1)
    in_sems = (sem_i0, sem_i1)
    out_sems = (sem_o0, sem_o1)

    def issue_in(db, b):
        base = pl.multiple_of(base_t + b * _EB2, 8)
        pltpu.make_async_copy(dst_hbm.at[pl.ds(base, _EB2)],
                              dsts[db].at[pl.ds(0, _EB2)],
                              in_sems[db]).start()
        pltpu.make_async_copy(esc_hbm.at[pl.ds(base * _H, _EB2 * _H)],
                              escs[db], in_sems[db]).start()

    def wait_in(db):
        pltpu.make_async_copy(dst_hbm.at[pl.ds(0, _EB2)],
                              dsts[db].at[pl.ds(0, _EB2)],
                              in_sems[db]).wait()
        pltpu.make_async_copy(esc_hbm.at[pl.ds(0, _EB2 * _H)], escs[db],
                              in_sems[db]).wait()

    def drain_out(db):
        pltpu.make_async_copy(nrms[db], out_hbm.at[pl.ds(0, _EB2 * _H)],
                              out_sems[db]).wait()

    def compute(db):
        dst_v = dsts[db]
        esc_v = escs[db]
        nrm_v = nrms[db]

        def pair(j, _):
            dv = dst_v[pl.ds(2 * j, 16)]
            d0 = dv[0]
            d1 = dv[1]
            row = jnp.where(msk8, d0 >> 4, d1 >> 4)
            colb = jnp.where(msk8, (d0 & 15) * _H, (d1 & 15) * _H)
            zi = plsc.load_gather(zv, [row, colb + lo3])
            sl = pl.ds(j * 16, 16)
            nrm_v[sl] = esc_v[sl] * zi
            return 0
        lax.fori_loop(0, _EB2 * _H // 16, pair, 0)

    def issue_out(db, b):
        base = pl.multiple_of(base_t + b * _EB2, 8)
        pltpu.make_async_copy(nrms[db],
                              out_hbm.at[pl.ds(base * _H, _EB2 * _H)],
                              out_sems[db]).start()

    issue_in(0, 0)

    def iter_body(g, _):
        for db in (0, 1):
            b = 2 * g + db
            nx = 1 - db

            @pl.when(b + 1 < _NBLK2)
            def _():
                issue_in(nx, b + 1)

            wait_in(db)

            @pl.when(b >= 2)
            def _():
                drain_out(db)

            compute(db)
            issue_out(db, b)
        return 0

    lax.fori_loop(0, _NBLK2 // 2, iter_body, 0)
    drain_out(0)
    drain_out(1)


def _norm_pass(esc, dst, zpart):
    mesh = plsc.VectorSubcoreMesh(core_axis_name="c", subcore_axis_name="s",
                                  num_cores=_NC, num_subcores=_NS)
    fn = pl.kernel(
        _norm_body,
        out_type=jax.ShapeDtypeStruct((_E * _H,), jnp.float32),
        mesh=mesh,
        scratch_types=[
            pltpu.VMEM((_ZR, _D), jnp.float32),
            pltpu.VMEM((_ZPT, _D), jnp.float32),
            pltpu.VMEM((_ZPT, _D), jnp.float32),
            pltpu.VMEM((_EB2 + 16,), jnp.int32),
            pltpu.VMEM((_EB2 + 16,), jnp.int32),
            pltpu.VMEM((_EB2 * _H,), jnp.float32),
            pltpu.VMEM((_EB2 * _H,), jnp.float32),
            pltpu.VMEM((_EB2 * _H,), jnp.float32),
            pltpu.VMEM((_EB2 * _H,), jnp.float32),
            pltpu.VMEM_SHARED((_ZR, _D), jnp.float32),
            pltpu.SemaphoreType.DMA,
            pltpu.SemaphoreType.DMA,
            pltpu.SemaphoreType.DMA,
            pltpu.SemaphoreType.DMA,
        ],
        compiler_params=pltpu.CompilerParams(needs_layout_passes=False),
    )
    return fn(esc, dst, zpart)


def kernel(x, y, edge_index, Wq, bq, Wk, bk, Wv, bv, We, be, Wr, br,
           Wg, bg, ln_g, ln_b):
    ca, cb = _cols_ab()
    wcat = jnp.concatenate([Wq[:, ca], Wq[:, cb], Wk[:, ca], Wk[:, cb],
                            Wv[:, ca], Wv[:, cb], Wr], axis=1)
    bcat = jnp.concatenate([bq[ca], bq[cb], bk[ca], bk[cb], bv[ca], bv[cb],
                            br]).reshape(1, 4 * _D)
    src, dst = _split_ei(edge_index)
    q, kv, r = _proj(x, wcat, bcat)
    e = _eproj(y, jnp.concatenate([We[:, ca], We[:, cb]], axis=1),
               jnp.concatenate([be[ca], be[cb]]).reshape(1, _D))
    zero = jnp.zeros((_RPT, _D), jnp.float32)
    esc, opart, zpart = _edge_pass(q, kv, e, src, dst, zero)
    wgh = Wg[0:_D] + Wg[2 * _D:3 * _D]
    wgr = Wg[_D:2 * _D] - Wg[2 * _D:3 * _D]
    out = _finalize(opart, zpart.reshape(_NC, _NP, _H), r, wgh, wgr,
                    bg.reshape(1, _D), ln_g.reshape(1, _D),
                    ln_b.reshape(1, _D))
    nrm = _norm_pass(esc, dst, zpart)
    return out, nrm.reshape(_E, _H, 1)
